# Initial kernel scaffold; baseline (speedup 1.0000x reference)
import jax
import jax.numpy as jnp
from jax.experimental import pallas as pl


def kernel(feat, edge_index, query, node_weight, node_bias, src_key_weight, dst_key_weight, src_key_bias, dst_key_bias, src_value_weight, dst_value_weight, src_value_bias, dst_value_bias, ln_gamma, ln_beta):
    raise NotImplementedError("write your pallas kernel here")


# speedup vs baseline: 4.8323x; 4.8323x over previous
import functools

import jax
import jax.numpy as jnp
from jax import lax
from jax.experimental import pallas as pl
from jax.experimental.pallas import tpu as pltpu

N, E, H, DH, D = 10000, 50000, 4, 4, 16
HD = H * DH

BE = 1000
BN = 1000


def _group_sum_matrix(rows, group):
    j = lax.broadcasted_iota(jnp.int32, (rows, rows // group), 0)
    o = lax.broadcasted_iota(jnp.int32, (rows, rows // group), 1)
    return (j // group == o).astype(jnp.float32)


def _group_sum_repl_matrix(n, group):
    i = lax.broadcasted_iota(jnp.int32, (n, n), 0)
    j = lax.broadcasted_iota(jnp.int32, (n, n), 1)
    return (i // group == j // group).astype(jnp.float32)


def _edgewise_body(skw, dkw, svw, dvw, skb, dkb, svb, dvb, fu, fv, qd,
                   key_o, val_o, exr_o):
    fur = jnp.concatenate([fu[...]] * HD, axis=1)
    fvr = jnp.concatenate([fv[...]] * HD, axis=1)
    S = _group_sum_matrix(D * HD, D)
    kp = skw[...] * fur + dkw[...] * fvr
    key = jnp.maximum(
        jnp.dot(kp, S, preferred_element_type=jnp.float32)
        + skb[...] + dkb[...], 0.0)
    vp = svw[...] * fur + dvw[...] * fvr
    val = jnp.maximum(
        jnp.dot(vp, S, preferred_element_type=jnp.float32)
        + svb[...] + dvb[...], 0.0)
    key_o[...] = key
    val_o[...] = val
    G = _group_sum_repl_matrix(HD, DH)
    lr = jnp.dot(key * qd[...], G, preferred_element_type=jnp.float32)
    exr_o[...] = jnp.exp(lr)


def _edgewise(skw, dkw, svw, dvw, skb, dkb, svb, dvb, fu, fv, qd):
    grid = E // BE
    wspec = pl.BlockSpec((BE, D * HD), lambda i: (i, 0))
    vspec = pl.BlockSpec((BE, HD), lambda i: (i, 0))
    return pl.pallas_call(
        _edgewise_body,
        grid=(grid,),
        in_specs=[wspec] * 4 + [vspec] * 7,
        out_specs=[vspec, vspec, vspec],
        out_shape=[jax.ShapeDtypeStruct((E, HD), jnp.float32)] * 3,
    )(skw, dkw, svw, dvw, skb, dkb, svb, dvb, fu, fv, qd)


def _nodewise_body(nw, nb, agg0, agg1, feat, g8, b8, out_o):
    agg = agg0[...] + agg1[...]
    ar = jnp.concatenate([agg] * HD, axis=1)
    S = _group_sum_matrix(D * D, D)
    pre = jnp.dot(nw[...] * ar, S, preferred_element_type=jnp.float32) + nb[...]
    o = jnp.maximum(pre, 0.0) + feat[...]
    mu = jnp.mean(o, axis=1, keepdims=True)
    dlt = o - mu
    var = jnp.mean(dlt * dlt, axis=1, keepdims=True)
    out_o[...] = dlt * lax.rsqrt(var + 1e-5) * g8[0:1, :] + b8[0:1, :]


def _nodewise(node_weight, node_bias, agg0, agg1, feat, ln_gamma, ln_beta):
    grid = N // BN
    wspec = pl.BlockSpec((BN, D * D), lambda i: (i, 0))
    vspec = pl.BlockSpec((BN, D), lambda i: (i, 0))
    gspec = pl.BlockSpec((8, D), lambda i: (0, 0))
    g8 = jnp.broadcast_to(ln_gamma.reshape(1, D), (8, D))
    b8 = jnp.broadcast_to(ln_beta.reshape(1, D), (8, D))
    return pl.pallas_call(
        _nodewise_body,
        grid=(grid,),
        in_specs=[wspec, vspec, vspec, vspec, vspec, gspec, gspec],
        out_specs=vspec,
        out_shape=jax.ShapeDtypeStruct((N, D), jnp.float32),
    )(node_weight, node_bias, agg0, agg1, feat, g8, b8)


def kernel(feat, edge_index, query, node_weight, node_bias,
           src_key_weight, dst_key_weight, src_key_bias, dst_key_bias,
           src_value_weight, dst_value_weight, src_value_bias, dst_value_bias,
           ln_gamma, ln_beta):
    src, dst = edge_index[0], edge_index[1]
    fu = jnp.take(feat, src, axis=0)
    fv = jnp.take(feat, dst, axis=0)
    qd = jnp.take(query.reshape(N, HD), dst, axis=0)

    skw = src_key_weight.reshape(E, HD * D)
    dkw = dst_key_weight.reshape(E, HD * D)
    svw = src_value_weight.reshape(E, HD * D)
    dvw = dst_value_weight.reshape(E, HD * D)
    skb = src_key_bias.reshape(E, HD)
    dkb = dst_key_bias.reshape(E, HD)
    svb = src_value_bias.reshape(E, HD)
    dvb = dst_value_bias.reshape(E, HD)

    key_e, val_e, exr = _edgewise(skw, dkw, svw, dvw, skb, dkb, svb, dvb,
                                  fu, fv, qd)

    denr = jax.ops.segment_sum(exr, dst, num_segments=N)
    ratio = exr / (jnp.take(denr, dst, axis=0) + 1e-16)
    msg = val_e * ratio
    agg = jax.ops.segment_sum(msg, dst, num_segments=N)
    attn = ratio.reshape(E, H, DH)[:, :, 0]

    out = _nodewise(node_weight.reshape(N, D * D), node_bias, agg,
                    jnp.zeros_like(agg), feat, ln_gamma, ln_beta)
    return (out, key_e, val_e, attn)


# speedup vs baseline: 8.5385x; 1.7670x over previous
import functools

import jax
import jax.numpy as jnp
from jax import lax
from jax.experimental import pallas as pl
from jax.experimental.pallas import tpu as pltpu
from jax.experimental.pallas import tpu_sc as plsc

N, E, H, DH, D = 10000, 50000, 4, 4, 16
HD = H * DH

BE = 1000
BN = 1000

NC, NS = 2, 16
NW = NC * NS
CHUNK = 128
K = 13
KP = 16
BPW = K * CHUNK
E_PAD = NW * BPW
N_PAD = 10240
NPS = N_PAD // NS

_mesh = plsc.VectorSubcoreMesh(core_axis_name="c", subcore_axis_name="s")
_sc_params = pltpu.CompilerParams(use_tc_tiling_on_sc=False)
_f32 = jnp.float32


@functools.partial(
    pl.kernel,
    out_type=[jax.ShapeDtypeStruct((E_PAD, D), _f32)] * 3,
    mesh=_mesh,
    compiler_params=_sc_params,
    scratch_types=[
        pltpu.VMEM((KP, CHUNK), jnp.int32),
        pltpu.VMEM((KP, CHUNK), jnp.int32),
        pltpu.VMEM((BPW, D), _f32),
        pltpu.VMEM((BPW, D), _f32),
        pltpu.VMEM((BPW, D), _f32),
        pltpu.SemaphoreType.DMA,
        pltpu.SemaphoreType.DMA,
        pltpu.SemaphoreType.DMA,
    ],
)
def _sc_gather(src3, dst3, feat_hbm, query_hbm, fu_o, fv_o, qd_o,
               sidx, didx, fub, fvb, qdb, sem0, sem1, sem2):
    wid = lax.axis_index("s") * NC + lax.axis_index("c")
    pltpu.sync_copy(src3.at[wid], sidx)
    pltpu.sync_copy(dst3.at[wid], didx)

    @pl.loop(0, K)
    def _chunks(j):
        c0 = pltpu.async_copy(feat_hbm.at[sidx.at[j]],
                              fub.at[pl.ds(j * CHUNK, CHUNK)], sem0)
        c1 = pltpu.async_copy(feat_hbm.at[didx.at[j]],
                              fvb.at[pl.ds(j * CHUNK, CHUNK)], sem1)
        c2 = pltpu.async_copy(query_hbm.at[didx.at[j]],
                              qdb.at[pl.ds(j * CHUNK, CHUNK)], sem2)
        c0.wait()
        c1.wait()
        c2.wait()

    base = wid * BPW
    pltpu.sync_copy(fub, fu_o.at[pl.ds(base, BPW)])
    pltpu.sync_copy(fvb, fv_o.at[pl.ds(base, BPW)])
    pltpu.sync_copy(qdb, qd_o.at[pl.ds(base, BPW)])


@functools.partial(
    pl.kernel,
    out_type=jax.ShapeDtypeStruct((NC * N_PAD, HD), _f32),
    mesh=_mesh,
    compiler_params=_sc_params,
    scratch_types=[
        pltpu.VMEM((KP, CHUNK), jnp.int32),
        pltpu.VMEM((BPW, HD), _f32),
        pltpu.VMEM((NPS, HD), _f32),
        pltpu.VMEM_SHARED((N_PAD, HD), _f32),
    ],
)
def _sc_denom(dst3, exr_hbm, den_o, didx, exb, zb, den_sh):
    cid = lax.axis_index("c")
    sid = lax.axis_index("s")
    wid = sid * NC + cid
    base = wid * BPW
    pltpu.sync_copy(dst3.at[wid], didx)
    pltpu.sync_copy(exr_hbm.at[pl.ds(base, BPW)], exb)

    @pl.loop(0, NPS)
    def _zero(i):
        zb[i, :] = jnp.zeros((HD,), _f32)

    pltpu.sync_copy(zb, den_sh.at[pl.ds(sid * NPS, NPS)])
    plsc.subcore_barrier()

    @pl.loop(0, K)
    def _scat(j):
        pltpu.sync_copy(exb.at[pl.ds(j * CHUNK, CHUNK)],
                        den_sh.at[didx.at[j]], add=True)

    plsc.subcore_barrier()
    pltpu.sync_copy(den_sh.at[pl.ds(sid * NPS, NPS)], zb)
    pltpu.sync_copy(zb, den_o.at[pl.ds(cid * N_PAD + sid * NPS, NPS)])


@functools.partial(
    pl.kernel,
    out_type=[jax.ShapeDtypeStruct((E_PAD, HD), _f32),
              jax.ShapeDtypeStruct((NC * N_PAD, HD), _f32)],
    mesh=_mesh,
    compiler_params=_sc_params,
    scratch_types=[
        pltpu.VMEM((KP, CHUNK), jnp.int32),
        pltpu.VMEM((BPW, HD), _f32),
        pltpu.VMEM((BPW, HD), _f32),
        pltpu.VMEM((BPW, HD), _f32),
        pltpu.VMEM((BPW, HD), _f32),
        pltpu.VMEM((NPS, HD), _f32),
        pltpu.VMEM_SHARED((N_PAD, HD), _f32),
        pltpu.SemaphoreType.DMA,
        pltpu.SemaphoreType.DMA,
    ],
)
def _sc_normalize_agg(dst3, exr_hbm, val_hbm, den0, den1, ratio_o, agg_o,
                      didx, exb, vb, g0, g1, zb, agg_sh, sem0, sem1):
    cid = lax.axis_index("c")
    sid = lax.axis_index("s")
    wid = sid * NC + cid
    base = wid * BPW
    pltpu.sync_copy(dst3.at[wid], didx)
    pltpu.sync_copy(exr_hbm.at[pl.ds(base, BPW)], exb)
    pltpu.sync_copy(val_hbm.at[pl.ds(base, BPW)], vb)

    @pl.loop(0, NPS)
    def _zero(i):
        zb[i, :] = jnp.zeros((HD,), _f32)

    pltpu.sync_copy(zb, agg_sh.at[pl.ds(sid * NPS, NPS)])

    @pl.loop(0, K)
    def _gath(j):
        c0 = pltpu.async_copy(den0.at[didx.at[j]],
                              g0.at[pl.ds(j * CHUNK, CHUNK)], sem0)
        c1 = pltpu.async_copy(den1.at[didx.at[j]],
                              g1.at[pl.ds(j * CHUNK, CHUNK)], sem1)
        c0.wait()
        c1.wait()

    @pl.loop(0, BPW)
    def _norm(i):
        den = g0[i, :] + g1[i, :]
        rt = exb[i, :] / (den + 1e-16)
        exb[i, :] = rt
        vb[i, :] = vb[i, :] * rt

    pltpu.sync_copy(exb, ratio_o.at[pl.ds(base, BPW)])
    plsc.subcore_barrier()

    @pl.loop(0, K)
    def _scat(j):
        pltpu.sync_copy(vb.at[pl.ds(j * CHUNK, CHUNK)],
                        agg_sh.at[didx.at[j]], add=True)

    plsc.subcore_barrier()
    pltpu.sync_copy(agg_sh.at[pl.ds(sid * NPS, NPS)], zb)
    pltpu.sync_copy(zb, agg_o.at[pl.ds(cid * N_PAD + sid * NPS, NPS)])


def _group_sum_matrix(rows, group):
    j = lax.broadcasted_iota(jnp.int32, (rows, rows // group), 0)
    o = lax.broadcasted_iota(jnp.int32, (rows, rows // group), 1)
    return (j // group == o).astype(_f32)


def _group_sum_repl_matrix(n, group):
    i = lax.broadcasted_iota(jnp.int32, (n, n), 0)
    j = lax.broadcasted_iota(jnp.int32, (n, n), 1)
    return (i // group == j // group).astype(_f32)


def _edgewise_body(skw, dkw, svw, dvw, skb, dkb, svb, dvb, fu, fv, qd,
                   key_o, val_o, exr_o):
    fur = jnp.concatenate([fu[...]] * HD, axis=1)
    fvr = jnp.concatenate([fv[...]] * HD, axis=1)
    S = _group_sum_matrix(D * HD, D)
    kp = skw[...] * fur + dkw[...] * fvr
    key = jnp.maximum(
        jnp.dot(kp, S, preferred_element_type=_f32) + skb[...] + dkb[...],
        0.0)
    vp = svw[...] * fur + dvw[...] * fvr
    val = jnp.maximum(
        jnp.dot(vp, S, preferred_element_type=_f32) + svb[...] + dvb[...],
        0.0)
    key_o[...] = key
    val_o[...] = val
    G = _group_sum_repl_matrix(HD, DH)
    lr = jnp.dot(key * qd[...], G, preferred_element_type=_f32)
    exr_o[...] = jnp.exp(lr)


def _edgewise(skw, dkw, svw, dvw, skb, dkb, svb, dvb, fu, fv, qd):
    wspec = pl.BlockSpec((BE, D * HD), lambda i: (i, 0))
    vspec = pl.BlockSpec((BE, HD), lambda i: (i, 0))
    return pl.pallas_call(
        _edgewise_body,
        grid=(E // BE,),
        in_specs=[wspec] * 4 + [vspec] * 7,
        out_specs=[vspec, vspec, vspec],
        out_shape=[jax.ShapeDtypeStruct((E, HD), _f32)] * 3,
    )(skw, dkw, svw, dvw, skb, dkb, svb, dvb, fu, fv, qd)


def _nodewise_body(nw, nb, agg0, agg1, feat, g8, b8, out_o):
    agg = agg0[...] + agg1[...]
    ar = jnp.concatenate([agg] * HD, axis=1)
    S = _group_sum_matrix(D * D, D)
    pre = jnp.dot(nw[...] * ar, S, preferred_element_type=_f32) + nb[...]
    o = jnp.maximum(pre, 0.0) + feat[...]
    mu = jnp.mean(o, axis=1, keepdims=True)
    dlt = o - mu
    var = jnp.mean(dlt * dlt, axis=1, keepdims=True)
    out_o[...] = dlt * lax.rsqrt(var + 1e-5) * g8[0:1, :] + b8[0:1, :]


def _nodewise(node_weight, node_bias, agg0, agg1, feat, ln_gamma, ln_beta):
    wspec = pl.BlockSpec((BN, D * D), lambda i: (i, 0))
    vspec = pl.BlockSpec((BN, D), lambda i: (i, 0))
    gspec = pl.BlockSpec((8, D), lambda i: (0, 0))
    g8 = jnp.broadcast_to(ln_gamma.reshape(1, D), (8, D))
    b8 = jnp.broadcast_to(ln_beta.reshape(1, D), (8, D))
    return pl.pallas_call(
        _nodewise_body,
        grid=(N // BN,),
        in_specs=[wspec, vspec, vspec, vspec, vspec, gspec, gspec],
        out_specs=vspec,
        out_shape=jax.ShapeDtypeStruct((N, D), _f32),
    )(node_weight, node_bias, agg0, agg1, feat, g8, b8)


def kernel(feat, edge_index, query, node_weight, node_bias,
           src_key_weight, dst_key_weight, src_key_bias, dst_key_bias,
           src_value_weight, dst_value_weight, src_value_bias, dst_value_bias,
           ln_gamma, ln_beta):
    src = jnp.pad(
        jnp.pad(edge_index[0], (0, E_PAD - E)).reshape(NW, K, CHUNK),
        ((0, 0), (0, KP - K), (0, 0)))
    dst = jnp.pad(
        jnp.pad(edge_index[1], (0, E_PAD - E)).reshape(NW, K, CHUNK),
        ((0, 0), (0, KP - K), (0, 0)))

    fu_p, fv_p, qd_p = _sc_gather(src, dst, feat, query.reshape(N, HD))
    fu, fv, qd = fu_p[:E], fv_p[:E], qd_p[:E]

    key_e, val_e, exr = _edgewise(
        src_key_weight.reshape(E, HD * D), dst_key_weight.reshape(E, HD * D),
        src_value_weight.reshape(E, HD * D), dst_value_weight.reshape(E, HD * D),
        src_key_bias.reshape(E, HD), dst_key_bias.reshape(E, HD),
        src_value_bias.reshape(E, HD), dst_value_bias.reshape(E, HD),
        fu, fv, qd)

    exr_p = jnp.pad(exr, ((0, E_PAD - E), (0, 0)))
    val_p = jnp.pad(val_e, ((0, E_PAD - E), (0, 0)))

    den01 = _sc_denom(dst, exr_p)
    ratio_p, agg01 = _sc_normalize_agg(dst, exr_p, val_p,
                                       den01[:N_PAD], den01[N_PAD:])

    attn = ratio_p[:E].reshape(E, H, DH)[:, :, 0]
    out = _nodewise(node_weight.reshape(N, D * D), node_bias,
                    agg01[:N], agg01[N_PAD:N_PAD + N], feat,
                    ln_gamma, ln_beta)
    return (out, key_e, val_e, attn)


# speedup vs baseline: 14.9975x; 1.7565x over previous
import functools

import jax
import jax.numpy as jnp
from jax import lax
from jax.experimental import pallas as pl
from jax.experimental.pallas import tpu as pltpu
from jax.experimental.pallas import tpu_sc as plsc

N, E, H, DH, D = 10000, 50000, 4, 4, 16
HD = H * DH

BE = 1024
BN = 2048

NC, NS = 2, 16
NW = NC * NS
CHUNK = 128
K = 13
KP = 16
BPW = K * CHUNK
SPAN2 = 2 * BPW
E_PAD = NW * BPW
N_PAD = 16384
NPS = N_PAD // NS

_mesh = plsc.VectorSubcoreMesh(core_axis_name="c", subcore_axis_name="s",
                               num_cores=NC, num_subcores=NS)
_sc_params = pltpu.CompilerParams(use_tc_tiling_on_sc=False)
_f32 = jnp.float32


@functools.partial(
    pl.kernel,
    out_type=[jax.ShapeDtypeStruct((E_PAD, D), _f32)] * 3,
    mesh=_mesh,
    compiler_params=_sc_params,
    scratch_types=[
        pltpu.VMEM((KP, CHUNK), jnp.int32),
        pltpu.VMEM((KP, CHUNK), jnp.int32),
        pltpu.VMEM((BPW, D), _f32),
        pltpu.VMEM((BPW, D), _f32),
        pltpu.VMEM((BPW, D), _f32),
        pltpu.SemaphoreType.DMA,
        pltpu.SemaphoreType.DMA,
        pltpu.SemaphoreType.DMA,
    ],
)
def _sc_gather(src3, dst3, feat_hbm, query_hbm, fu_o, fv_o, qd_o,
               sidx, didx, fub, fvb, qdb, sem0, sem1, sem2):
    wid = lax.axis_index("s") * NC + lax.axis_index("c")
    pltpu.sync_copy(src3.at[wid], sidx)
    pltpu.sync_copy(dst3.at[wid], didx)

    def _start(j):
        pltpu.async_copy(feat_hbm.at[sidx.at[j]],
                         fub.at[pl.ds(j * CHUNK, CHUNK)], sem0)
        pltpu.async_copy(feat_hbm.at[didx.at[j]],
                         fvb.at[pl.ds(j * CHUNK, CHUNK)], sem1)
        pltpu.async_copy(query_hbm.at[didx.at[j]],
                         qdb.at[pl.ds(j * CHUNK, CHUNK)], sem2)

    def _wait(j):
        pltpu.make_async_copy(feat_hbm.at[sidx.at[j]],
                              fub.at[pl.ds(j * CHUNK, CHUNK)], sem0).wait()
        pltpu.make_async_copy(feat_hbm.at[didx.at[j]],
                              fvb.at[pl.ds(j * CHUNK, CHUNK)], sem1).wait()
        pltpu.make_async_copy(query_hbm.at[didx.at[j]],
                              qdb.at[pl.ds(j * CHUNK, CHUNK)], sem2).wait()

    @pl.loop(0, K)
    def _chunks(j):
        _start(j)

        @pl.when(j > 0)
        def _():
            _wait(j - 1)

    _wait(K - 1)
    base = wid * BPW
    pltpu.sync_copy(fub, fu_o.at[pl.ds(base, BPW)])
    pltpu.sync_copy(fvb, fv_o.at[pl.ds(base, BPW)])
    pltpu.sync_copy(qdb, qd_o.at[pl.ds(base, BPW)])


@functools.partial(
    pl.kernel,
    out_type=[jax.ShapeDtypeStruct((E_PAD, HD), _f32),
              jax.ShapeDtypeStruct((NC * N_PAD, HD), _f32)],
    mesh=_mesh,
    compiler_params=_sc_params,
    scratch_types=[
        pltpu.VMEM((2, KP, CHUNK), jnp.int32),
        pltpu.VMEM((BPW, HD), _f32),
        pltpu.VMEM((BPW, HD), _f32),
        pltpu.VMEM((BPW, HD), _f32),
        pltpu.VMEM((2, CHUNK, HD), _f32),
        pltpu.VMEM((NPS, HD), _f32),
        pltpu.VMEM_SHARED((N_PAD, HD), _f32),
        pltpu.SemaphoreType.DMA,
        pltpu.SemaphoreType.DMA,
    ],
)
def _sc_softmax_agg(dst3, exr_hbm, val_hbm, ratio_o, agg_o,
                    didx, exb, vb, g0, sb, zb, den_sh,
                    sem_s, sem_g):
    cid = lax.axis_index("c")
    sid = lax.axis_index("s")
    base_own = sid * SPAN2 + cid * BPW
    base_c = sid * SPAN2 + (1 - cid) * BPW
    pltpu.sync_copy(dst3.at[pl.ds(2 * sid, 2)], didx)
    pltpu.sync_copy(exr_hbm.at[pl.ds(base_own, BPW)], exb)
    pltpu.sync_copy(val_hbm.at[pl.ds(base_own, BPW)], vb)

    @pl.loop(0, NPS, unroll=4)
    def _zero(i):
        zb[i, :] = jnp.zeros((HD,), _f32)

    pltpu.sync_copy(zb, den_sh.at[pl.ds(sid * NPS, NPS)])
    plsc.subcore_barrier()

    @pl.loop(0, K)
    def _scat_own(j):
        pltpu.sync_copy(exb.at[pl.ds(j * CHUNK, CHUNK)],
                        den_sh.at[didx.at[cid, j]], add=True)

    def _cstart(j):
        pltpu.async_copy(exr_hbm.at[pl.ds(base_c + j * CHUNK, CHUNK)],
                         sb.at[j % 2], sem_s)

    def _cdone(j):
        pltpu.make_async_copy(exr_hbm.at[pl.ds(base_c + j * CHUNK, CHUNK)],
                              sb.at[j % 2], sem_s).wait()
        pltpu.sync_copy(sb.at[j % 2], den_sh.at[didx.at[1 - cid, j]],
                        add=True)

    @pl.loop(0, K)
    def _scat_compl(j):
        _cstart(j)

        @pl.when(j > 0)
        def _():
            _cdone(j - 1)

    _cdone(K - 1)
    plsc.subcore_barrier()

    def _gstart(j):
        pltpu.async_copy(den_sh.at[didx.at[cid, j]],
                         g0.at[pl.ds(j * CHUNK, CHUNK)], sem_g)

    def _gwait(j):
        pltpu.make_async_copy(den_sh.at[didx.at[cid, j]],
                              g0.at[pl.ds(j * CHUNK, CHUNK)], sem_g).wait()

    @pl.loop(0, K)
    def _gath(j):
        _gstart(j)

        @pl.when(j > 0)
        def _():
            _gwait(j - 1)

    _gwait(K - 1)
    plsc.subcore_barrier()
    pltpu.sync_copy(zb, den_sh.at[pl.ds(sid * NPS, NPS)])
    plsc.subcore_barrier()

    @pl.loop(0, BPW, unroll=4)
    def _norm(i):
        rt = exb[i, :] / (g0[i, :] + 1e-16)
        vb[i, :] = vb[i, :] * rt
        exb[i, :] = rt

    pltpu.sync_copy(exb, ratio_o.at[pl.ds(base_own, BPW)])

    @pl.loop(0, K)
    def _scat_msg(j):
        pltpu.sync_copy(vb.at[pl.ds(j * CHUNK, CHUNK)],
                        den_sh.at[didx.at[cid, j]], add=True)

    plsc.subcore_barrier()
    pltpu.sync_copy(den_sh.at[pl.ds(sid * NPS, NPS)], zb)
    pltpu.sync_copy(zb, agg_o.at[pl.ds(cid * N_PAD + sid * NPS, NPS)])


def _group_sum_matrix_t(rows, group):
    o = lax.broadcasted_iota(jnp.int32, (rows // group, rows), 0)
    j = lax.broadcasted_iota(jnp.int32, (rows // group, rows), 1)
    return (j // group == o).astype(_f32)


def _group_sum_repl_matrix(n, group):
    i = lax.broadcasted_iota(jnp.int32, (n, n), 0)
    j = lax.broadcasted_iota(jnp.int32, (n, n), 1)
    return (i // group == j // group).astype(_f32)


def _edgewise_body(skw, dkw, svw, dvw, skb, dkb, svb, dvb, fu, fv, qd,
                   key_o, val_o, exr_o):
    fur = jnp.concatenate([fu[...]] * HD, axis=0)
    fvr = jnp.concatenate([fv[...]] * HD, axis=0)
    S = _group_sum_matrix_t(D * HD, D)
    kp = skw[...] * fur + dkw[...] * fvr
    key = jnp.maximum(
        jnp.dot(S, kp, preferred_element_type=_f32) + skb[...] + dkb[...],
        0.0)
    vp = svw[...] * fur + dvw[...] * fvr
    val = jnp.maximum(
        jnp.dot(S, vp, preferred_element_type=_f32) + svb[...] + dvb[...],
        0.0)
    key_o[...] = key
    val_o[...] = val
    G = _group_sum_repl_matrix(HD, DH)
    lr = jnp.dot(G, key * qd[...], preferred_element_type=_f32)
    exr_o[...] = jnp.exp(lr)


def _edgewise(skw, dkw, svw, dvw, skb, dkb, svb, dvb, fu, fv, qd):
    wspec = pl.BlockSpec((D * HD, BE), lambda i: (0, i))
    vspec = pl.BlockSpec((HD, BE), lambda i: (0, i))
    grid = (E + BE - 1) // BE
    return pl.pallas_call(
        _edgewise_body,
        grid=(grid,),
        in_specs=[wspec] * 4 + [vspec] * 7,
        out_specs=[vspec, vspec, vspec],
        out_shape=[jax.ShapeDtypeStruct((HD, E), _f32)] * 3,
    )(skw, dkw, svw, dvw, skb, dkb, svb, dvb, fu, fv, qd)


def _nodewise_body(nw, nb, agg0, agg1, feat, g2, out_o):
    agg = agg0[...] + agg1[...]
    ar = jnp.concatenate([agg] * D, axis=0)
    S = _group_sum_matrix_t(D * D, D)
    pre = jnp.dot(S, nw[...] * ar, preferred_element_type=_f32) + nb[...]
    o = jnp.maximum(pre, 0.0) + feat[...]
    mu = jnp.mean(o, axis=0, keepdims=True)
    dlt = o - mu
    var = jnp.mean(dlt * dlt, axis=0, keepdims=True)
    gb = g2[...]
    out_o[...] = dlt * lax.rsqrt(var + 1e-5) * gb[:, 0:1] + gb[:, 1:2]


def _nodewise(nw_t, nb_t, agg01_t, feat_t, ln_gamma, ln_beta):
    wspec = pl.BlockSpec((D * D, BN), lambda i: (0, i))
    vspec = pl.BlockSpec((D, BN), lambda i: (0, i))
    a1spec = pl.BlockSpec((D, BN), lambda i: (0, i + N_PAD // BN))
    gspec = pl.BlockSpec((D, 128), lambda i: (0, 0))
    g2 = jnp.zeros((D, 128), _f32)
    g2 = g2.at[:, 0].set(ln_gamma).at[:, 1].set(ln_beta)
    return pl.pallas_call(
        _nodewise_body,
        grid=((N + BN - 1) // BN,),
        in_specs=[wspec, vspec, vspec, a1spec, vspec, gspec],
        out_specs=vspec,
        out_shape=jax.ShapeDtypeStruct((D, N), _f32),
    )(nw_t, nb_t, agg01_t, agg01_t, feat_t, g2)


def kernel(feat, edge_index, query, node_weight, node_bias,
           src_key_weight, dst_key_weight, src_key_bias, dst_key_bias,
           src_value_weight, dst_value_weight, src_value_bias, dst_value_bias,
           ln_gamma, ln_beta):
    src = jnp.pad(
        jnp.pad(edge_index[0], (0, E_PAD - E)).reshape(NW, K, CHUNK),
        ((0, 0), (0, KP - K), (0, 0)))
    dst = jnp.pad(
        jnp.pad(edge_index[1], (0, E_PAD - E), constant_values=N)
        .reshape(NW, K, CHUNK),
        ((0, 0), (0, KP - K), (0, 0)))

    fu_p, fv_p, qd_p = _sc_gather(src, dst, feat, query.reshape(N, HD))

    def wv(w):
        return w.transpose(1, 2, 3, 0).reshape(HD * D, E)

    def bv(b):
        return b.transpose(1, 2, 0).reshape(HD, E)

    key_t, val_t, exr_t = _edgewise(
        wv(src_key_weight), wv(dst_key_weight),
        wv(src_value_weight), wv(dst_value_weight),
        bv(src_key_bias), bv(dst_key_bias),
        bv(src_value_bias), bv(dst_value_bias),
        fu_p.T, fv_p.T, qd_p.T)

    exr_p = jnp.pad(exr_t.T, ((0, E_PAD - E), (0, 0)))
    val_p = jnp.pad(val_t.T, ((0, E_PAD - E), (0, 0)))

    ratio_p, agg01 = _sc_softmax_agg(dst, exr_p, val_p)
    attn = ratio_p[:E].reshape(E, H, DH)[:, :, 0]

    out_t = _nodewise(node_weight.transpose(1, 2, 0).reshape(D * D, N),
                      node_bias.T, agg01.T, feat.T, ln_gamma, ln_beta)
    return (out_t.T, key_t.T, val_t.T, attn)


# speedup vs baseline: 18.0645x; 1.2045x over previous
import functools

import jax
import jax.numpy as jnp
from jax import lax
from jax.experimental import pallas as pl
from jax.experimental.pallas import tpu as pltpu
from jax.experimental.pallas import tpu_sc as plsc

N, E, H, DH, D = 10000, 50000, 4, 4, 16
HD = H * DH

BE = 1024
BN = 2048

NC, NS = 2, 16
NW = NC * NS
CHUNK = 128
K = 13
KP = 16
BPW = K * CHUNK
SPAN2 = 2 * BPW
E_PAD = NW * BPW
N_PAD = 16384
NPS = N_PAD // NS

_mesh = plsc.VectorSubcoreMesh(core_axis_name="c", subcore_axis_name="s",
                               num_cores=NC, num_subcores=NS)
_sc_params = pltpu.CompilerParams(use_tc_tiling_on_sc=False)
_f32 = jnp.float32


@functools.partial(
    pl.kernel,
    out_type=[jax.ShapeDtypeStruct((E_PAD, D), _f32)] * 3,
    mesh=_mesh,
    compiler_params=_sc_params,
    scratch_types=[
        pltpu.VMEM((KP, CHUNK), jnp.int32),
        pltpu.VMEM((KP, CHUNK), jnp.int32),
        pltpu.VMEM((BPW, D), _f32),
        pltpu.VMEM((BPW, D), _f32),
        pltpu.VMEM((BPW, D), _f32),
        pltpu.SemaphoreType.DMA,
        pltpu.SemaphoreType.DMA,
        pltpu.SemaphoreType.DMA,
    ],
)
def _sc_gather(src3, dst3, feat_hbm, query_hbm, fu_o, fv_o, qd_o,
               sidx, didx, fub, fvb, qdb, sem0, sem1, sem2):
    wid = lax.axis_index("s") * NC + lax.axis_index("c")
    pltpu.sync_copy(src3.at[wid], sidx)
    pltpu.sync_copy(dst3.at[wid], didx)

    def _start(j):
        pltpu.async_copy(feat_hbm.at[sidx.at[j]],
                         fub.at[pl.ds(j * CHUNK, CHUNK)], sem0)
        pltpu.async_copy(feat_hbm.at[didx.at[j]],
                         fvb.at[pl.ds(j * CHUNK, CHUNK)], sem1)
        pltpu.async_copy(query_hbm.at[didx.at[j]],
                         qdb.at[pl.ds(j * CHUNK, CHUNK)], sem2)

    def _wait(j):
        pltpu.make_async_copy(feat_hbm.at[sidx.at[j]],
                              fub.at[pl.ds(j * CHUNK, CHUNK)], sem0).wait()
        pltpu.make_async_copy(feat_hbm.at[didx.at[j]],
                              fvb.at[pl.ds(j * CHUNK, CHUNK)], sem1).wait()
        pltpu.make_async_copy(query_hbm.at[didx.at[j]],
                              qdb.at[pl.ds(j * CHUNK, CHUNK)], sem2).wait()

    @pl.loop(0, K)
    def _chunks(j):
        _start(j)

        @pl.when(j > 0)
        def _():
            _wait(j - 1)

    _wait(K - 1)
    base = wid * BPW
    pltpu.sync_copy(fub, fu_o.at[pl.ds(base, BPW)])
    pltpu.sync_copy(fvb, fv_o.at[pl.ds(base, BPW)])
    pltpu.sync_copy(qdb, qd_o.at[pl.ds(base, BPW)])


@functools.partial(
    pl.kernel,
    out_type=[jax.ShapeDtypeStruct((E_PAD, HD), _f32),
              jax.ShapeDtypeStruct((NC * N_PAD, HD), _f32)],
    mesh=_mesh,
    compiler_params=_sc_params,
    scratch_types=[
        pltpu.VMEM((2, KP, CHUNK), jnp.int32),
        pltpu.VMEM((BPW, HD), _f32),
        pltpu.VMEM((BPW, HD), _f32),
        pltpu.VMEM((BPW, HD), _f32),
        pltpu.VMEM((2, CHUNK, HD), _f32),
        pltpu.VMEM((NPS, HD), _f32),
        pltpu.VMEM_SHARED((N_PAD, HD), _f32),
        pltpu.SemaphoreType.DMA,
        pltpu.SemaphoreType.DMA,
    ],
)
def _sc_softmax_agg(dst3, exr_hbm, val_hbm, ratio_o, agg_o,
                    didx, exb, vb, g0, sb, zb, den_sh,
                    sem_s, sem_g):
    cid = lax.axis_index("c")
    sid = lax.axis_index("s")
    base_own = sid * SPAN2 + cid * BPW
    base_c = sid * SPAN2 + (1 - cid) * BPW
    pltpu.sync_copy(dst3.at[pl.ds(2 * sid, 2)], didx)
    pltpu.sync_copy(exr_hbm.at[pl.ds(base_own, BPW)], exb)
    pltpu.sync_copy(val_hbm.at[pl.ds(base_own, BPW)], vb)

    @pl.loop(0, NPS, unroll=4)
    def _zero(i):
        zb[i, :] = jnp.zeros((HD,), _f32)

    pltpu.sync_copy(zb, den_sh.at[pl.ds(sid * NPS, NPS)])
    plsc.subcore_barrier()

    @pl.loop(0, K)
    def _scat_own(j):
        pltpu.sync_copy(exb.at[pl.ds(j * CHUNK, CHUNK)],
                        den_sh.at[didx.at[cid, j]], add=True)

    def _cstart(j):
        pltpu.async_copy(exr_hbm.at[pl.ds(base_c + j * CHUNK, CHUNK)],
                         sb.at[j % 2], sem_s)

    def _cdone(j):
        pltpu.make_async_copy(exr_hbm.at[pl.ds(base_c + j * CHUNK, CHUNK)],
                              sb.at[j % 2], sem_s).wait()
        pltpu.sync_copy(sb.at[j % 2], den_sh.at[didx.at[1 - cid, j]],
                        add=True)

    @pl.loop(0, K)
    def _scat_compl(j):
        _cstart(j)

        @pl.when(j > 0)
        def _():
            _cdone(j - 1)

    _cdone(K - 1)
    plsc.subcore_barrier()

    def _gstart(j):
        pltpu.async_copy(den_sh.at[didx.at[cid, j]],
                         g0.at[pl.ds(j * CHUNK, CHUNK)], sem_g)

    def _gwait(j):
        pltpu.make_async_copy(den_sh.at[didx.at[cid, j]],
                              g0.at[pl.ds(j * CHUNK, CHUNK)], sem_g).wait()

    @pl.loop(0, K)
    def _gath(j):
        _gstart(j)

        @pl.when(j > 0)
        def _():
            _gwait(j - 1)

    _gwait(K - 1)
    plsc.subcore_barrier()
    pltpu.sync_copy(zb, den_sh.at[pl.ds(sid * NPS, NPS)])
    plsc.subcore_barrier()

    @pl.loop(0, BPW, unroll=4)
    def _norm(i):
        rt = exb[i, :] / (g0[i, :] + 1e-16)
        vb[i, :] = vb[i, :] * rt
        exb[i, :] = rt

    pltpu.sync_copy(exb, ratio_o.at[pl.ds(base_own, BPW)])

    @pl.loop(0, K)
    def _scat_msg(j):
        pltpu.sync_copy(vb.at[pl.ds(j * CHUNK, CHUNK)],
                        den_sh.at[didx.at[cid, j]], add=True)

    plsc.subcore_barrier()
    pltpu.sync_copy(den_sh.at[pl.ds(sid * NPS, NPS)], zb)
    pltpu.sync_copy(zb, agg_o.at[pl.ds(cid * N_PAD + sid * NPS, NPS)])


def _group_sum_matrix_t(rows, group):
    o = lax.broadcasted_iota(jnp.int32, (rows // group, rows), 0)
    j = lax.broadcasted_iota(jnp.int32, (rows // group, rows), 1)
    return (j // group == o).astype(_f32)


def _group_sum_repl_matrix(n, group):
    i = lax.broadcasted_iota(jnp.int32, (n, n), 0)
    j = lax.broadcasted_iota(jnp.int32, (n, n), 1)
    return (i // group == j // group).astype(_f32)


def _edgewise_body(skw, dkw, svw, dvw, skb, dkb, svb, dvb, fu, fv, qd,
                   key_o, val_o, exr_o, val2_o):
    fut = fu[...].T
    fvt = fv[...].T
    qdt = qd[...].T
    fur = jnp.concatenate([fut] * HD, axis=0)
    fvr = jnp.concatenate([fvt] * HD, axis=0)
    S = _group_sum_matrix_t(D * HD, D)
    kp = skw[...] * fur + dkw[...] * fvr
    key = jnp.maximum(
        jnp.dot(S, kp, preferred_element_type=_f32) + skb[...] + dkb[...],
        0.0)
    vp = svw[...] * fur + dvw[...] * fvr
    val = jnp.maximum(
        jnp.dot(S, vp, preferred_element_type=_f32) + svb[...] + dvb[...],
        0.0)
    key_o[...] = key
    val_o[...] = val
    val2_o[...] = val.T
    G = _group_sum_repl_matrix(HD, DH)
    lr = jnp.dot(G, key * qdt, preferred_element_type=_f32)
    exr_o[...] = jnp.exp(lr).T


def _edgewise(skw, dkw, svw, dvw, skb, dkb, svb, dvb, fu, fv, qd):
    wspec = pl.BlockSpec((D * HD, BE), lambda i: (0, i))
    vspec = pl.BlockSpec((HD, BE), lambda i: (0, i))
    rspec = pl.BlockSpec((BE, HD), lambda i: (i, 0))
    grid = (E + BE - 1) // BE
    return pl.pallas_call(
        _edgewise_body,
        grid=(grid,),
        in_specs=[wspec] * 4 + [vspec] * 4 + [rspec] * 3,
        out_specs=[vspec, vspec, rspec, rspec],
        out_shape=[jax.ShapeDtypeStruct((HD, E), _f32)] * 2
        + [jax.ShapeDtypeStruct((E_PAD, HD), _f32)] * 2,
    )(skw, dkw, svw, dvw, skb, dkb, svb, dvb, fu, fv, qd)


def _nodewise_body(nw, nb, agg0, agg1, feat, g2, out_o):
    agg = (agg0[...] + agg1[...]).T
    ar = jnp.concatenate([agg] * D, axis=0)
    S = _group_sum_matrix_t(D * D, D)
    pre = jnp.dot(S, nw[...] * ar, preferred_element_type=_f32) + nb[...]
    o = jnp.maximum(pre, 0.0) + feat[...]
    mu = jnp.mean(o, axis=0, keepdims=True)
    dlt = o - mu
    var = jnp.mean(dlt * dlt, axis=0, keepdims=True)
    gb = g2[...]
    out_o[...] = dlt * lax.rsqrt(var + 1e-5) * gb[:, 0:1] + gb[:, 1:2]


def _nodewise(nw_t, nb_t, agg01_t, feat_t, ln_gamma, ln_beta):
    wspec = pl.BlockSpec((D * D, BN), lambda i: (0, i))
    vspec = pl.BlockSpec((D, BN), lambda i: (0, i))
    a0spec = pl.BlockSpec((BN, D), lambda i: (i, 0))
    a1spec = pl.BlockSpec((BN, D), lambda i: (i + N_PAD // BN, 0))
    gspec = pl.BlockSpec((D, 128), lambda i: (0, 0))
    g2 = jnp.zeros((D, 128), _f32)
    g2 = g2.at[:, 0].set(ln_gamma).at[:, 1].set(ln_beta)
    return pl.pallas_call(
        _nodewise_body,
        grid=((N + BN - 1) // BN,),
        in_specs=[wspec, vspec, a0spec, a1spec, vspec, gspec],
        out_specs=vspec,
        out_shape=jax.ShapeDtypeStruct((D, N), _f32),
    )(nw_t, nb_t, agg01_t, agg01_t, feat_t, g2)


def kernel(feat, edge_index, query, node_weight, node_bias,
           src_key_weight, dst_key_weight, src_key_bias, dst_key_bias,
           src_value_weight, dst_value_weight, src_value_bias, dst_value_bias,
           ln_gamma, ln_beta):
    src = jnp.pad(
        jnp.pad(edge_index[0], (0, E_PAD - E)).reshape(NW, K, CHUNK),
        ((0, 0), (0, KP - K), (0, 0)))
    dst = jnp.pad(
        jnp.pad(edge_index[1], (0, E_PAD - E), constant_values=N)
        .reshape(NW, K, CHUNK),
        ((0, 0), (0, KP - K), (0, 0)))

    fu_p, fv_p, qd_p = _sc_gather(src, dst, feat, query.reshape(N, HD))

    def wv(w):
        return w.transpose(1, 2, 3, 0).reshape(HD * D, E)

    def bv(b):
        return b.transpose(1, 2, 0).reshape(HD, E)

    key_t, val_t, exr_p, val_p = _edgewise(
        wv(src_key_weight), wv(dst_key_weight),
        wv(src_value_weight), wv(dst_value_weight),
        bv(src_key_bias), bv(dst_key_bias),
        bv(src_value_bias), bv(dst_value_bias),
        fu_p, fv_p, qd_p)

    ratio_p, agg01 = _sc_softmax_agg(dst, exr_p, val_p)
    attn = ratio_p[:E].reshape(E, H, DH)[:, :, 0]

    out_t = _nodewise(node_weight.transpose(1, 2, 0).reshape(D * D, N),
                      node_bias.T, agg01, feat.T, ln_gamma, ln_beta)
    return (out_t.T, key_t.T, val_t.T, attn)


# speedup vs baseline: 18.9867x; 1.0510x over previous
import functools

import jax
import jax.numpy as jnp
from jax import lax
from jax.experimental import pallas as pl
from jax.experimental.pallas import tpu as pltpu
from jax.experimental.pallas import tpu_sc as plsc

N, E, H, DH, D = 10000, 50000, 4, 4, 16
HD = H * DH

BE = 1024
BN = 2048

NC, NS = 2, 16
NW = NC * NS
CHUNK = 128
K = 13
KP = 16
BPW = K * CHUNK
SPAN2 = 2 * BPW
E_PAD = NW * BPW
N_PAD = 16384
NPS = N_PAD // NS

_mesh = plsc.VectorSubcoreMesh(core_axis_name="c", subcore_axis_name="s",
                               num_cores=NC, num_subcores=NS)
_sc_params = pltpu.CompilerParams(use_tc_tiling_on_sc=False,
                                 needs_layout_passes=False)
_f32 = jnp.float32


@functools.partial(
    pl.kernel,
    out_type=[jax.ShapeDtypeStruct((D, E_PAD), _f32)] * 3,
    mesh=_mesh,
    compiler_params=_sc_params,
    scratch_types=[
        pltpu.VMEM((KP, CHUNK), jnp.int32),
        pltpu.VMEM((KP, CHUNK), jnp.int32),
        pltpu.VMEM((BPW, D), _f32),
        pltpu.VMEM((BPW, D), _f32),
        pltpu.VMEM((BPW, D), _f32),
        pltpu.VMEM((D, BPW), _f32),
        pltpu.SemaphoreType.DMA,
        pltpu.SemaphoreType.DMA,
        pltpu.SemaphoreType.DMA,
    ],
)
def _sc_gather(src3, dst3, feat_hbm, query_hbm, fu_o, fv_o, qd_o,
               sidx, didx, fub, fvb, qdb, tb, sem0, sem1, sem2):
    wid = lax.axis_index("s") * NC + lax.axis_index("c")
    pltpu.sync_copy(src3.at[wid], sidx)
    pltpu.sync_copy(dst3.at[wid], didx)

    def _start(j):
        pltpu.async_copy(feat_hbm.at[sidx.at[j]],
                         fub.at[pl.ds(j * CHUNK, CHUNK)], sem0)
        pltpu.async_copy(feat_hbm.at[didx.at[j]],
                         fvb.at[pl.ds(j * CHUNK, CHUNK)], sem1)
        pltpu.async_copy(query_hbm.at[didx.at[j]],
                         qdb.at[pl.ds(j * CHUNK, CHUNK)], sem2)

    def _wait(j):
        pltpu.make_async_copy(feat_hbm.at[sidx.at[j]],
                              fub.at[pl.ds(j * CHUNK, CHUNK)], sem0).wait()
        pltpu.make_async_copy(feat_hbm.at[didx.at[j]],
                              fvb.at[pl.ds(j * CHUNK, CHUNK)], sem1).wait()
        pltpu.make_async_copy(query_hbm.at[didx.at[j]],
                              qdb.at[pl.ds(j * CHUNK, CHUNK)], sem2).wait()

    @pl.loop(0, K)
    def _chunks(j):
        _start(j)

        @pl.when(j > 0)
        def _():
            _wait(j - 1)

    _wait(K - 1)
    base = wid * BPW
    iot = lax.iota(jnp.int32, D)

    def _emit_t(buf, out):
        @pl.loop(0, BPW // D)
        def _tr(b):
            rows = b * D + iot
            for c in range(D):
                col = jnp.full((D,), c, jnp.int32)
                tb[c, pl.ds(b * D, D)] = plsc.load_gather(buf, [rows, col])
        for r in range(D):
            pltpu.sync_copy(tb.at[r], out.at[r, pl.ds(base, BPW)])

    _emit_t(fub, fu_o)
    _emit_t(fvb, fv_o)
    _emit_t(qdb, qd_o)


@functools.partial(
    pl.kernel,
    out_type=[jax.ShapeDtypeStruct((E_PAD, HD), _f32),
              jax.ShapeDtypeStruct((NC * N_PAD, HD), _f32)],
    mesh=_mesh,
    compiler_params=_sc_params,
    scratch_types=[
        pltpu.VMEM((2, KP, CHUNK), jnp.int32),
        pltpu.VMEM((BPW, HD), _f32),
        pltpu.VMEM((BPW, HD), _f32),
        pltpu.VMEM((BPW, HD), _f32),
        pltpu.VMEM((2, CHUNK, HD), _f32),
        pltpu.VMEM((NPS, HD), _f32),
        pltpu.VMEM_SHARED((N_PAD, HD), _f32),
        pltpu.SemaphoreType.DMA,
        pltpu.SemaphoreType.DMA,
    ],
)
def _sc_softmax_agg(dst3, exr_hbm, val_hbm, ratio_o, agg_o,
                    didx, exb, vb, g0, sb, zb, den_sh,
                    sem_s, sem_g):
    cid = lax.axis_index("c")
    sid = lax.axis_index("s")
    base_own = sid * SPAN2 + cid * BPW
    base_c = sid * SPAN2 + (1 - cid) * BPW
    pltpu.sync_copy(dst3.at[pl.ds(2 * sid, 2)], didx)
    pltpu.sync_copy(exr_hbm.at[pl.ds(base_own, BPW)], exb)
    pltpu.sync_copy(val_hbm.at[pl.ds(base_own, BPW)], vb)

    @pl.loop(0, NPS, unroll=4)
    def _zero(i):
        zb[i, :] = jnp.zeros((HD,), _f32)

    pltpu.sync_copy(zb, den_sh.at[pl.ds(sid * NPS, NPS)])
    plsc.subcore_barrier()

    @pl.loop(0, K)
    def _scat_own(j):
        pltpu.sync_copy(exb.at[pl.ds(j * CHUNK, CHUNK)],
                        den_sh.at[didx.at[cid, j]], add=True)

    def _cstart(j):
        pltpu.async_copy(exr_hbm.at[pl.ds(base_c + j * CHUNK, CHUNK)],
                         sb.at[j % 2], sem_s)

    def _cdone(j):
        pltpu.make_async_copy(exr_hbm.at[pl.ds(base_c + j * CHUNK, CHUNK)],
                              sb.at[j % 2], sem_s).wait()
        pltpu.sync_copy(sb.at[j % 2], den_sh.at[didx.at[1 - cid, j]],
                        add=True)

    @pl.loop(0, K)
    def _scat_compl(j):
        _cstart(j)

        @pl.when(j > 0)
        def _():
            _cdone(j - 1)

    _cdone(K - 1)
    plsc.subcore_barrier()

    def _gstart(j):
        pltpu.async_copy(den_sh.at[didx.at[cid, j]],
                         g0.at[pl.ds(j * CHUNK, CHUNK)], sem_g)

    def _gwait(j):
        pltpu.make_async_copy(den_sh.at[didx.at[cid, j]],
                              g0.at[pl.ds(j * CHUNK, CHUNK)], sem_g).wait()

    @pl.loop(0, K)
    def _gath(j):
        _gstart(j)

        @pl.when(j > 0)
        def _():
            _gwait(j - 1)

    _gwait(K - 1)
    plsc.subcore_barrier()
    pltpu.sync_copy(zb, den_sh.at[pl.ds(sid * NPS, NPS)])
    plsc.subcore_barrier()

    @pl.loop(0, BPW, unroll=4)
    def _norm(i):
        rt = exb[i, :] / (g0[i, :] + 1e-16)
        vb[i, :] = vb[i, :] * rt
        exb[i, :] = rt

    pltpu.sync_copy(exb, ratio_o.at[pl.ds(base_own, BPW)])

    @pl.loop(0, K)
    def _scat_msg(j):
        pltpu.sync_copy(vb.at[pl.ds(j * CHUNK, CHUNK)],
                        den_sh.at[didx.at[cid, j]], add=True)

    plsc.subcore_barrier()
    pltpu.sync_copy(den_sh.at[pl.ds(sid * NPS, NPS)], zb)
    pltpu.sync_copy(zb, agg_o.at[pl.ds(cid * N_PAD + sid * NPS, NPS)])


def _group_sum_matrix_t(rows, group):
    o = lax.broadcasted_iota(jnp.int32, (rows // group, rows), 0)
    j = lax.broadcasted_iota(jnp.int32, (rows // group, rows), 1)
    return (j // group == o).astype(_f32)


def _group_sum_repl_matrix(n, group):
    i = lax.broadcasted_iota(jnp.int32, (n, n), 0)
    j = lax.broadcasted_iota(jnp.int32, (n, n), 1)
    return (i // group == j // group).astype(_f32)


def _edgewise_body(skw, dkw, svw, dvw, skb, dkb, svb, dvb, fu, fv, qd,
                   key_o, val_o, exr_o, val2_o):
    fur = jnp.concatenate([fu[...]] * HD, axis=0)
    fvr = jnp.concatenate([fv[...]] * HD, axis=0)
    S = _group_sum_matrix_t(D * HD, D)
    kp = skw[...] * fur + dkw[...] * fvr
    key = jnp.maximum(
        jnp.dot(S, kp, preferred_element_type=_f32) + skb[...] + dkb[...],
        0.0)
    vp = svw[...] * fur + dvw[...] * fvr
    val = jnp.maximum(
        jnp.dot(S, vp, preferred_element_type=_f32) + svb[...] + dvb[...],
        0.0)
    key_o[...] = key
    val_o[...] = val
    val2_o[...] = val.T
    G = _group_sum_repl_matrix(HD, DH)
    lr = jnp.dot(G, key * qd[...], preferred_element_type=_f32)
    exr_o[...] = jnp.exp(lr).T


def _edgewise(skw, dkw, svw, dvw, skb, dkb, svb, dvb, fu, fv, qd):
    wspec = pl.BlockSpec((D * HD, BE), lambda i: (0, i))
    vspec = pl.BlockSpec((HD, BE), lambda i: (0, i))
    rspec = pl.BlockSpec((BE, HD), lambda i: (i, 0))
    grid = (E + BE - 1) // BE
    return pl.pallas_call(
        _edgewise_body,
        grid=(grid,),
        in_specs=[wspec] * 4 + [vspec] * 7,
        out_specs=[vspec, vspec, rspec, rspec],
        out_shape=[jax.ShapeDtypeStruct((HD, E), _f32)] * 2
        + [jax.ShapeDtypeStruct((E_PAD, HD), _f32)] * 2,
    )(skw, dkw, svw, dvw, skb, dkb, svb, dvb, fu, fv, qd)


def _nodewise_body(nw, nb, agg0, agg1, feat, g2, out_o):
    agg = (agg0[...] + agg1[...]).T
    ar = jnp.concatenate([agg] * D, axis=0)
    S = _group_sum_matrix_t(D * D, D)
    pre = jnp.dot(S, nw[...] * ar, preferred_element_type=_f32) + nb[...]
    o = jnp.maximum(pre, 0.0) + feat[...]
    mu = jnp.mean(o, axis=0, keepdims=True)
    dlt = o - mu
    var = jnp.mean(dlt * dlt, axis=0, keepdims=True)
    gb = g2[...]
    out_o[...] = dlt * lax.rsqrt(var + 1e-5) * gb[:, 0:1] + gb[:, 1:2]


def _nodewise(nw_t, nb_t, agg01_t, feat_t, ln_gamma, ln_beta):
    wspec = pl.BlockSpec((D * D, BN), lambda i: (0, i))
    vspec = pl.BlockSpec((D, BN), lambda i: (0, i))
    a0spec = pl.BlockSpec((BN, D), lambda i: (i, 0))
    a1spec = pl.BlockSpec((BN, D), lambda i: (i + N_PAD // BN, 0))
    gspec = pl.BlockSpec((D, 128), lambda i: (0, 0))
    g2 = jnp.zeros((D, 128), _f32)
    g2 = g2.at[:, 0].set(ln_gamma).at[:, 1].set(ln_beta)
    return pl.pallas_call(
        _nodewise_body,
        grid=((N + BN - 1) // BN,),
        in_specs=[wspec, vspec, a0spec, a1spec, vspec, gspec],
        out_specs=vspec,
        out_shape=jax.ShapeDtypeStruct((D, N), _f32),
    )(nw_t, nb_t, agg01_t, agg01_t, feat_t, g2)


def kernel(feat, edge_index, query, node_weight, node_bias,
           src_key_weight, dst_key_weight, src_key_bias, dst_key_bias,
           src_value_weight, dst_value_weight, src_value_bias, dst_value_bias,
           ln_gamma, ln_beta):
    src = jnp.pad(
        jnp.pad(edge_index[0], (0, E_PAD - E)).reshape(NW, K, CHUNK),
        ((0, 0), (0, KP - K), (0, 0)))
    dst = jnp.pad(
        jnp.pad(edge_index[1], (0, E_PAD - E), constant_values=N)
        .reshape(NW, K, CHUNK),
        ((0, 0), (0, KP - K), (0, 0)))

    fu_p, fv_p, qd_p = _sc_gather(src, dst, feat, query.reshape(N, HD))

    def wv(w):
        return w.transpose(1, 2, 3, 0).reshape(HD * D, E)

    def bv(b):
        return b.transpose(1, 2, 0).reshape(HD, E)

    key_t, val_t, exr_p, val_p = _edgewise(
        wv(src_key_weight), wv(dst_key_weight),
        wv(src_value_weight), wv(dst_value_weight),
        bv(src_key_bias), bv(dst_key_bias),
        bv(src_value_bias), bv(dst_value_bias),
        fu_p, fv_p, qd_p)

    ratio_p, agg01 = _sc_softmax_agg(dst, exr_p, val_p)
    attn = ratio_p[:E].reshape(E, H, DH)[:, :, 0]

    out_t = _nodewise(node_weight.transpose(1, 2, 0).reshape(D * D, N),
                      node_bias.T, agg01, feat.T, ln_gamma, ln_beta)
    return (out_t.T, key_t.T, val_t.T, attn)


# speedup vs baseline: 21.1394x; 1.1134x over previous
import functools

import jax
import jax.numpy as jnp
from jax import lax
from jax.experimental import pallas as pl
from jax.experimental.pallas import tpu as pltpu
from jax.experimental.pallas import tpu_sc as plsc

N, E, H, DH, D = 10000, 50000, 4, 4, 16
HD = H * DH

BE = 1024
BN = 2048

NC, NS = 2, 16
NW = NC * NS
CHUNK = 128
K = 13
KP = 16
BPW = K * CHUNK
SPAN2 = 2 * BPW
E_PAD = NW * BPW
N_PAD = 16384
NPS = N_PAD // NS

_mesh = plsc.VectorSubcoreMesh(core_axis_name="c", subcore_axis_name="s",
                               num_cores=NC, num_subcores=NS)
_sc_params = pltpu.CompilerParams(use_tc_tiling_on_sc=False,
                                 needs_layout_passes=False)
_f32 = jnp.float32


@functools.partial(
    pl.kernel,
    out_type=[jax.ShapeDtypeStruct((D, E_PAD), _f32)] * 3,
    mesh=_mesh,
    compiler_params=_sc_params,
    scratch_types=[
        pltpu.VMEM((KP, CHUNK), jnp.int32),
        pltpu.VMEM((KP, CHUNK), jnp.int32),
        pltpu.VMEM((BPW, D), _f32),
        pltpu.VMEM((BPW, D), _f32),
        pltpu.VMEM((BPW, D), _f32),
        pltpu.VMEM((D, BPW), _f32),
        pltpu.SemaphoreType.DMA,
        pltpu.SemaphoreType.DMA,
        pltpu.SemaphoreType.DMA,
    ],
)
def _sc_gather(src3, dst3, feat_hbm, query_hbm, fu_o, fv_o, qd_o,
               sidx, didx, fub, fvb, qdb, tb, sem0, sem1, sem2):
    wid = lax.axis_index("s") * NC + lax.axis_index("c")
    pltpu.sync_copy(src3.at[wid], sidx)
    pltpu.sync_copy(dst3.at[wid], didx)

    def _start(j):
        pltpu.async_copy(feat_hbm.at[sidx.at[j]],
                         fub.at[pl.ds(j * CHUNK, CHUNK)], sem0)
        pltpu.async_copy(feat_hbm.at[didx.at[j]],
                         fvb.at[pl.ds(j * CHUNK, CHUNK)], sem1)
        pltpu.async_copy(query_hbm.at[didx.at[j]],
                         qdb.at[pl.ds(j * CHUNK, CHUNK)], sem2)

    def _wait(j):
        pltpu.make_async_copy(feat_hbm.at[sidx.at[j]],
                              fub.at[pl.ds(j * CHUNK, CHUNK)], sem0).wait()
        pltpu.make_async_copy(feat_hbm.at[didx.at[j]],
                              fvb.at[pl.ds(j * CHUNK, CHUNK)], sem1).wait()
        pltpu.make_async_copy(query_hbm.at[didx.at[j]],
                              qdb.at[pl.ds(j * CHUNK, CHUNK)], sem2).wait()

    @pl.loop(0, K)
    def _chunks(j):
        _start(j)

        @pl.when(j > 0)
        def _():
            _wait(j - 1)

    _wait(K - 1)
    base = wid * BPW
    iot = lax.iota(jnp.int32, D)

    def _emit_t(buf, out):
        @pl.loop(0, BPW // D)
        def _tr(b):
            rows = b * D + iot
            for c in range(D):
                col = jnp.full((D,), c, jnp.int32)
                tb[c, pl.ds(b * D, D)] = plsc.load_gather(buf, [rows, col])
        for r in range(D):
            pltpu.sync_copy(tb.at[r], out.at[r, pl.ds(base, BPW)])

    _emit_t(fub, fu_o)
    _emit_t(fvb, fv_o)
    _emit_t(qdb, qd_o)


@functools.partial(
    pl.kernel,
    out_type=[jax.ShapeDtypeStruct((H, E_PAD), _f32),
              jax.ShapeDtypeStruct((D, NC * N_PAD), _f32)],
    mesh=_mesh,
    compiler_params=_sc_params,
    scratch_types=[
        pltpu.VMEM((2, KP, CHUNK), jnp.int32),
        pltpu.VMEM((HD, BPW), _f32),
        pltpu.VMEM((BPW, HD), _f32),
        pltpu.VMEM((BPW, HD), _f32),
        pltpu.VMEM((BPW, HD), _f32),
        pltpu.VMEM((CHUNK, HD), _f32),
        pltpu.VMEM_SHARED((N_PAD, HD), _f32),
        pltpu.SemaphoreType.DMA,
        pltpu.SemaphoreType.DMA,
    ],
)
def _sc_softmax_agg(dst3, exr_hbm, val_hbm, attn_o, agg_o,
                    didx, slab, exb, vb, g0, sbt, den_sh,
                    sem_s, sem_g):
    cid = lax.axis_index("c")
    sid = lax.axis_index("s")
    base_own = sid * SPAN2 + cid * BPW
    base_c = sid * SPAN2 + (1 - cid) * BPW
    pltpu.sync_copy(dst3.at[pl.ds(2 * sid, 2)], didx)

    iot = lax.iota(jnp.int32, HD)

    def _slab_load(src, base):
        for r in range(HD):
            pltpu.async_copy(src.at[r, pl.ds(base, BPW)], slab.at[r], sem_s)
        for r in range(HD):
            pltpu.make_async_copy(src.at[r, pl.ds(base, BPW)], slab.at[r],
                                  sem_s).wait()

    def _slab_to_rows(dst):
        @pl.loop(0, BPW // HD)
        def _tr(b):
            for r in range(HD):
                col = jnp.full((HD,), b * HD + r, jnp.int32)
                dst[b * HD + r, :] = plsc.load_gather(slab, [iot, col])

    def _rows_to_slab(src, ncols):
        @pl.loop(0, ncols // HD)
        def _tr(b):
            rows = b * HD + iot
            for c in range(HD):
                col = jnp.full((HD,), c, jnp.int32)
                slab[c, pl.ds(b * HD, HD)] = plsc.load_gather(src, [rows, col])

    _slab_load(exr_hbm, base_own)
    _slab_to_rows(exb)
    _slab_load(val_hbm, base_own)
    _slab_to_rows(vb)

    @pl.loop(0, CHUNK, unroll=4)
    def _zero(i):
        sbt[i, :] = jnp.zeros((HD,), _f32)

    for t in range(NPS // CHUNK):
        pltpu.sync_copy(sbt, den_sh.at[pl.ds(sid * NPS + t * CHUNK, CHUNK)])
    plsc.subcore_barrier()

    @pl.loop(0, K)
    def _scat_own(j):
        pltpu.sync_copy(exb.at[pl.ds(j * CHUNK, CHUNK)],
                        den_sh.at[didx.at[cid, j]], add=True)

    _slab_load(exr_hbm, base_c)

    @pl.loop(0, K)
    def _scat_compl(j):
        @pl.loop(0, CHUNK // HD)
        def _tr(b):
            for r in range(HD):
                col = jnp.full((HD,), j * CHUNK + b * HD + r, jnp.int32)
                sbt[b * HD + r, :] = plsc.load_gather(slab, [iot, col])

        pltpu.sync_copy(sbt, den_sh.at[didx.at[1 - cid, j]], add=True)

    plsc.subcore_barrier()

    def _gstart(j):
        pltpu.async_copy(den_sh.at[didx.at[cid, j]],
                         g0.at[pl.ds(j * CHUNK, CHUNK)], sem_g)

    def _gwait(j):
        pltpu.make_async_copy(den_sh.at[didx.at[cid, j]],
                              g0.at[pl.ds(j * CHUNK, CHUNK)], sem_g).wait()

    @pl.loop(0, K)
    def _gath(j):
        _gstart(j)

        @pl.when(j > 0)
        def _():
            _gwait(j - 1)

    _gwait(K - 1)
    plsc.subcore_barrier()

    @pl.loop(0, CHUNK, unroll=4)
    def _zero2(i):
        sbt[i, :] = jnp.zeros((HD,), _f32)

    for t in range(NPS // CHUNK):
        pltpu.sync_copy(sbt, den_sh.at[pl.ds(sid * NPS + t * CHUNK, CHUNK)])
    plsc.subcore_barrier()

    @pl.loop(0, BPW, unroll=4)
    def _norm(i):
        rt = exb[i, :] / (g0[i, :] + 1e-16)
        vb[i, :] = vb[i, :] * rt
        exb[i, :] = rt

    @pl.loop(0, BPW // HD)
    def _trat(b):
        rows = b * HD + iot
        for h in range(H):
            col = jnp.full((HD,), h * DH, jnp.int32)
            slab[h, pl.ds(b * HD, HD)] = plsc.load_gather(exb, [rows, col])

    for r in range(H):
        pltpu.async_copy(slab.at[r], attn_o.at[r, pl.ds(base_own, BPW)],
                         sem_s)
    for r in range(H):
        pltpu.make_async_copy(slab.at[r], attn_o.at[r, pl.ds(base_own, BPW)],
                              sem_s).wait()

    @pl.loop(0, K)
    def _scat_msg(j):
        pltpu.sync_copy(vb.at[pl.ds(j * CHUNK, CHUNK)],
                        den_sh.at[didx.at[cid, j]], add=True)

    plsc.subcore_barrier()
    pltpu.sync_copy(den_sh.at[pl.ds(sid * NPS, NPS)], exb.at[pl.ds(0, NPS)])
    _rows_to_slab(exb, NPS)
    abase = cid * N_PAD + sid * NPS
    for r in range(D):
        pltpu.async_copy(slab.at[r, pl.ds(0, NPS)],
                         agg_o.at[r, pl.ds(abase, NPS)], sem_s)
    for r in range(D):
        pltpu.make_async_copy(slab.at[r, pl.ds(0, NPS)],
                              agg_o.at[r, pl.ds(abase, NPS)], sem_s).wait()


def _group_sum_matrix_t(rows, group):
    o = lax.broadcasted_iota(jnp.int32, (rows // group, rows), 0)
    j = lax.broadcasted_iota(jnp.int32, (rows // group, rows), 1)
    return (j // group == o).astype(_f32)


def _group_sum_repl_matrix(n, group):
    i = lax.broadcasted_iota(jnp.int32, (n, n), 0)
    j = lax.broadcasted_iota(jnp.int32, (n, n), 1)
    return (i // group == j // group).astype(_f32)


def _edgewise_body(skw, dkw, svw, dvw, skb, dkb, svb, dvb, fu, fv, qd,
                   key_o, val_o, exr_o, val2_o):
    fur = jnp.concatenate([fu[...]] * HD, axis=0)
    fvr = jnp.concatenate([fv[...]] * HD, axis=0)
    S = _group_sum_matrix_t(D * HD, D)
    kp = skw[...] * fur + dkw[...] * fvr
    key = jnp.maximum(
        jnp.dot(S, kp, preferred_element_type=_f32) + skb[...] + dkb[...],
        0.0)
    vp = svw[...] * fur + dvw[...] * fvr
    val = jnp.maximum(
        jnp.dot(S, vp, preferred_element_type=_f32) + svb[...] + dvb[...],
        0.0)
    key_o[...] = key
    val_o[...] = val
    val2_o[...] = val
    G = _group_sum_repl_matrix(HD, DH)
    lr = jnp.dot(G, key * qd[...], preferred_element_type=_f32)
    exr_o[...] = jnp.exp(lr)


def _edgewise(skw, dkw, svw, dvw, skb, dkb, svb, dvb, fu, fv, qd):
    wspec = pl.BlockSpec((D * HD, BE), lambda i: (0, i))
    vspec = pl.BlockSpec((HD, BE), lambda i: (0, i))
    grid = (E + BE - 1) // BE
    return pl.pallas_call(
        _edgewise_body,
        grid=(grid,),
        in_specs=[wspec] * 4 + [vspec] * 7,
        out_specs=[vspec] * 4,
        out_shape=[jax.ShapeDtypeStruct((HD, E), _f32)] * 2
        + [jax.ShapeDtypeStruct((HD, E_PAD), _f32)] * 2,
    )(skw, dkw, svw, dvw, skb, dkb, svb, dvb, fu, fv, qd)


def _nodewise_body(nw, nb, agg0, agg1, feat, g2, out_o):
    agg = agg0[...] + agg1[...]
    ar = jnp.concatenate([agg] * D, axis=0)
    S = _group_sum_matrix_t(D * D, D)
    pre = jnp.dot(S, nw[...] * ar, preferred_element_type=_f32) + nb[...]
    o = jnp.maximum(pre, 0.0) + feat[...]
    mu = jnp.mean(o, axis=0, keepdims=True)
    dlt = o - mu
    var = jnp.mean(dlt * dlt, axis=0, keepdims=True)
    gb = g2[...]
    out_o[...] = dlt * lax.rsqrt(var + 1e-5) * gb[:, 0:1] + gb[:, 1:2]


def _nodewise(nw_t, nb_t, agg01_t, feat_t, ln_gamma, ln_beta):
    wspec = pl.BlockSpec((D * D, BN), lambda i: (0, i))
    vspec = pl.BlockSpec((D, BN), lambda i: (0, i))
    a0spec = pl.BlockSpec((D, BN), lambda i: (0, i))
    a1spec = pl.BlockSpec((D, BN), lambda i: (0, i + N_PAD // BN))
    gspec = pl.BlockSpec((D, 128), lambda i: (0, 0))
    g2 = jnp.zeros((D, 128), _f32)
    g2 = g2.at[:, 0].set(ln_gamma).at[:, 1].set(ln_beta)
    return pl.pallas_call(
        _nodewise_body,
        grid=((N + BN - 1) // BN,),
        in_specs=[wspec, vspec, a0spec, a1spec, vspec, gspec],
        out_specs=vspec,
        out_shape=jax.ShapeDtypeStruct((D, N), _f32),
    )(nw_t, nb_t, agg01_t, agg01_t, feat_t, g2)


def kernel(feat, edge_index, query, node_weight, node_bias,
           src_key_weight, dst_key_weight, src_key_bias, dst_key_bias,
           src_value_weight, dst_value_weight, src_value_bias, dst_value_bias,
           ln_gamma, ln_beta):
    src = jnp.pad(
        jnp.pad(edge_index[0], (0, E_PAD - E)).reshape(NW, K, CHUNK),
        ((0, 0), (0, KP - K), (0, 0)))
    dst = jnp.pad(
        jnp.pad(edge_index[1], (0, E_PAD - E), constant_values=N)
        .reshape(NW, K, CHUNK),
        ((0, 0), (0, KP - K), (0, 0)))

    fu_p, fv_p, qd_p = _sc_gather(src, dst, feat, query.reshape(N, HD))

    def wv(w):
        return w.transpose(1, 2, 3, 0).reshape(HD * D, E)

    def bv(b):
        return b.transpose(1, 2, 0).reshape(HD, E)

    key_t, val_t, exr_p, val_p = _edgewise(
        wv(src_key_weight), wv(dst_key_weight),
        wv(src_value_weight), wv(dst_value_weight),
        bv(src_key_bias), bv(dst_key_bias),
        bv(src_value_bias), bv(dst_value_bias),
        fu_p, fv_p, qd_p)

    attn_t, agg01 = _sc_softmax_agg(dst, exr_p, val_p)
    attn = attn_t[:, :E].T

    out_t = _nodewise(node_weight.transpose(1, 2, 0).reshape(D * D, N),
                      node_bias.T, agg01, feat.T, ln_gamma, ln_beta)
    return (out_t.T, key_t.T, val_t.T, attn)


# speedup vs baseline: 21.2226x; 1.0039x over previous
import functools

import jax
import jax.numpy as jnp
from jax import lax
from jax.experimental import pallas as pl
from jax.experimental.pallas import tpu as pltpu
from jax.experimental.pallas import tpu_sc as plsc

N, E, H, DH, D = 10000, 50000, 4, 4, 16
HD = H * DH

BE = 2048
BN = 2048

NC, NS = 2, 16
NW = NC * NS
CHUNK = 128
K = 13
KP = 16
BPW = K * CHUNK
SPAN2 = 2 * BPW
E_PAD = NW * BPW
N_PAD = 16384
NPS = N_PAD // NS

_mesh = plsc.VectorSubcoreMesh(core_axis_name="c", subcore_axis_name="s",
                               num_cores=NC, num_subcores=NS)
_sc_params = pltpu.CompilerParams(use_tc_tiling_on_sc=False,
                                 needs_layout_passes=False)
_f32 = jnp.float32


@functools.partial(
    pl.kernel,
    out_type=[jax.ShapeDtypeStruct((D, E_PAD), _f32)] * 3,
    mesh=_mesh,
    compiler_params=_sc_params,
    scratch_types=[
        pltpu.VMEM((KP, CHUNK), jnp.int32),
        pltpu.VMEM((KP, CHUNK), jnp.int32),
        pltpu.VMEM((BPW, D), _f32),
        pltpu.VMEM((BPW, D), _f32),
        pltpu.VMEM((BPW, D), _f32),
        pltpu.VMEM((D, BPW), _f32),
        pltpu.SemaphoreType.DMA,
        pltpu.SemaphoreType.DMA,
        pltpu.SemaphoreType.DMA,
    ],
)
def _sc_gather(src3, dst3, feat_hbm, query_hbm, fu_o, fv_o, qd_o,
               sidx, didx, fub, fvb, qdb, tb, sem0, sem1, sem2):
    wid = lax.axis_index("s") * NC + lax.axis_index("c")
    pltpu.sync_copy(src3.at[wid], sidx)
    pltpu.sync_copy(dst3.at[wid], didx)

    def _start(j):
        pltpu.async_copy(feat_hbm.at[sidx.at[j]],
                         fub.at[pl.ds(j * CHUNK, CHUNK)], sem0)
        pltpu.async_copy(feat_hbm.at[didx.at[j]],
                         fvb.at[pl.ds(j * CHUNK, CHUNK)], sem1)
        pltpu.async_copy(query_hbm.at[didx.at[j]],
                         qdb.at[pl.ds(j * CHUNK, CHUNK)], sem2)

    def _wait(j):
        pltpu.make_async_copy(feat_hbm.at[sidx.at[j]],
                              fub.at[pl.ds(j * CHUNK, CHUNK)], sem0).wait()
        pltpu.make_async_copy(feat_hbm.at[didx.at[j]],
                              fvb.at[pl.ds(j * CHUNK, CHUNK)], sem1).wait()
        pltpu.make_async_copy(query_hbm.at[didx.at[j]],
                              qdb.at[pl.ds(j * CHUNK, CHUNK)], sem2).wait()

    @pl.loop(0, K)
    def _chunks(j):
        _start(j)

        @pl.when(j > 0)
        def _():
            _wait(j - 1)

    _wait(K - 1)
    base = wid * BPW
    iot = lax.iota(jnp.int32, D)

    def _emit_t(buf, out):
        @pl.loop(0, BPW // D, unroll=2)
        def _tr(b):
            rows = b * D + iot
            for c in range(D):
                col = jnp.full((D,), c, jnp.int32)
                tb[c, pl.ds(b * D, D)] = plsc.load_gather(buf, [rows, col])
        for r in range(D):
            pltpu.sync_copy(tb.at[r], out.at[r, pl.ds(base, BPW)])

    _emit_t(fub, fu_o)
    _emit_t(fvb, fv_o)
    _emit_t(qdb, qd_o)


@functools.partial(
    pl.kernel,
    out_type=[jax.ShapeDtypeStruct((H, E_PAD), _f32),
              jax.ShapeDtypeStruct((D, NC * N_PAD), _f32)],
    mesh=_mesh,
    compiler_params=_sc_params,
    scratch_types=[
        pltpu.VMEM((2, KP, CHUNK), jnp.int32),
        pltpu.VMEM((HD, BPW), _f32),
        pltpu.VMEM((BPW, HD), _f32),
        pltpu.VMEM((BPW, HD), _f32),
        pltpu.VMEM((BPW, HD), _f32),
        pltpu.VMEM((CHUNK, HD), _f32),
        pltpu.VMEM_SHARED((N_PAD, HD), _f32),
        pltpu.SemaphoreType.DMA,
        pltpu.SemaphoreType.DMA,
    ],
)
def _sc_softmax_agg(dst3, exr_hbm, val_hbm, attn_o, agg_o,
                    didx, slab, exb, vb, g0, sbt, den_sh,
                    sem_s, sem_g):
    cid = lax.axis_index("c")
    sid = lax.axis_index("s")
    base_own = sid * SPAN2 + cid * BPW
    base_c = sid * SPAN2 + (1 - cid) * BPW
    pltpu.sync_copy(dst3.at[pl.ds(2 * sid, 2)], didx)

    iot = lax.iota(jnp.int32, HD)

    def _slab_load(src, base):
        for r in range(HD):
            pltpu.async_copy(src.at[r, pl.ds(base, BPW)], slab.at[r], sem_s)
        for r in range(HD):
            pltpu.make_async_copy(src.at[r, pl.ds(base, BPW)], slab.at[r],
                                  sem_s).wait()

    def _slab_to_rows(dst):
        @pl.loop(0, BPW // HD, unroll=2)
        def _tr(b):
            for r in range(HD):
                col = jnp.full((HD,), b * HD + r, jnp.int32)
                dst[b * HD + r, :] = plsc.load_gather(slab, [iot, col])

    def _rows_to_slab(src, ncols):
        @pl.loop(0, ncols // HD, unroll=2)
        def _tr(b):
            rows = b * HD + iot
            for c in range(HD):
                col = jnp.full((HD,), c, jnp.int32)
                slab[c, pl.ds(b * HD, HD)] = plsc.load_gather(src, [rows, col])

    _slab_load(exr_hbm, base_own)
    _slab_to_rows(exb)
    _slab_load(val_hbm, base_own)
    _slab_to_rows(vb)

    @pl.loop(0, CHUNK, unroll=4)
    def _zero(i):
        sbt[i, :] = jnp.zeros((HD,), _f32)

    for t in range(NPS // CHUNK):
        pltpu.sync_copy(sbt, den_sh.at[pl.ds(sid * NPS + t * CHUNK, CHUNK)])
    plsc.subcore_barrier()

    @pl.loop(0, K)
    def _scat_own(j):
        pltpu.sync_copy(exb.at[pl.ds(j * CHUNK, CHUNK)],
                        den_sh.at[didx.at[cid, j]], add=True)

    _slab_load(exr_hbm, base_c)

    @pl.loop(0, K)
    def _scat_compl(j):
        @pl.loop(0, CHUNK // HD)
        def _tr(b):
            for r in range(HD):
                col = jnp.full((HD,), j * CHUNK + b * HD + r, jnp.int32)
                sbt[b * HD + r, :] = plsc.load_gather(slab, [iot, col])

        pltpu.sync_copy(sbt, den_sh.at[didx.at[1 - cid, j]], add=True)

    plsc.subcore_barrier()

    def _gstart(j):
        pltpu.async_copy(den_sh.at[didx.at[cid, j]],
                         g0.at[pl.ds(j * CHUNK, CHUNK)], sem_g)

    def _gwait(j):
        pltpu.make_async_copy(den_sh.at[didx.at[cid, j]],
                              g0.at[pl.ds(j * CHUNK, CHUNK)], sem_g).wait()

    @pl.loop(0, K)
    def _gath(j):
        _gstart(j)

        @pl.when(j > 0)
        def _():
            _gwait(j - 1)

    _gwait(K - 1)
    plsc.subcore_barrier()

    @pl.loop(0, CHUNK, unroll=4)
    def _zero2(i):
        sbt[i, :] = jnp.zeros((HD,), _f32)

    for t in range(NPS // CHUNK):
        pltpu.sync_copy(sbt, den_sh.at[pl.ds(sid * NPS + t * CHUNK, CHUNK)])
    plsc.subcore_barrier()

    @pl.loop(0, BPW, unroll=4)
    def _norm(i):
        rt = exb[i, :] / (g0[i, :] + 1e-16)
        vb[i, :] = vb[i, :] * rt
        exb[i, :] = rt

    @pl.loop(0, BPW // HD, unroll=2)
    def _trat(b):
        rows = b * HD + iot
        for h in range(H):
            col = jnp.full((HD,), h * DH, jnp.int32)
            slab[h, pl.ds(b * HD, HD)] = plsc.load_gather(exb, [rows, col])

    for r in range(H):
        pltpu.async_copy(slab.at[r], attn_o.at[r, pl.ds(base_own, BPW)],
                         sem_s)
    for r in range(H):
        pltpu.make_async_copy(slab.at[r], attn_o.at[r, pl.ds(base_own, BPW)],
                              sem_s).wait()

    @pl.loop(0, K)
    def _scat_msg(j):
        pltpu.sync_copy(vb.at[pl.ds(j * CHUNK, CHUNK)],
                        den_sh.at[didx.at[cid, j]], add=True)

    plsc.subcore_barrier()
    pltpu.sync_copy(den_sh.at[pl.ds(sid * NPS, NPS)], exb.at[pl.ds(0, NPS)])
    _rows_to_slab(exb, NPS)
    abase = cid * N_PAD + sid * NPS
    for r in range(D):
        pltpu.async_copy(slab.at[r, pl.ds(0, NPS)],
                         agg_o.at[r, pl.ds(abase, NPS)], sem_s)
    for r in range(D):
        pltpu.make_async_copy(slab.at[r, pl.ds(0, NPS)],
                              agg_o.at[r, pl.ds(abase, NPS)], sem_s).wait()


def _group_sum_matrix_t(rows, group):
    o = lax.broadcasted_iota(jnp.int32, (rows // group, rows), 0)
    j = lax.broadcasted_iota(jnp.int32, (rows // group, rows), 1)
    return (j // group == o).astype(_f32)


def _group_sum_repl_matrix(n, group):
    i = lax.broadcasted_iota(jnp.int32, (n, n), 0)
    j = lax.broadcasted_iota(jnp.int32, (n, n), 1)
    return (i // group == j // group).astype(_f32)


def _edgewise_body(skw, dkw, svw, dvw, skb, dkb, svb, dvb, fu, fv, qd,
                   key_o, val_o, exr_o, val2_o):
    fur = jnp.concatenate([fu[...]] * HD, axis=0)
    fvr = jnp.concatenate([fv[...]] * HD, axis=0)
    S = _group_sum_matrix_t(D * HD, D)
    kp = skw[...] * fur + dkw[...] * fvr
    key = jnp.maximum(
        jnp.dot(S, kp, preferred_element_type=_f32) + skb[...] + dkb[...],
        0.0)
    vp = svw[...] * fur + dvw[...] * fvr
    val = jnp.maximum(
        jnp.dot(S, vp, preferred_element_type=_f32) + svb[...] + dvb[...],
        0.0)
    key_o[...] = key
    val_o[...] = val
    val2_o[...] = val
    G = _group_sum_repl_matrix(HD, DH)
    lr = jnp.dot(G, key * qd[...], preferred_element_type=_f32)
    exr_o[...] = jnp.exp(lr)


def _edgewise(skw, dkw, svw, dvw, skb, dkb, svb, dvb, fu, fv, qd):
    wspec = pl.BlockSpec((D * HD, BE), lambda i: (0, i))
    vspec = pl.BlockSpec((HD, BE), lambda i: (0, i))
    grid = (E + BE - 1) // BE
    return pl.pallas_call(
        _edgewise_body,
        grid=(grid,),
        in_specs=[wspec] * 4 + [vspec] * 7,
        out_specs=[vspec] * 4,
        out_shape=[jax.ShapeDtypeStruct((HD, E), _f32)] * 2
        + [jax.ShapeDtypeStruct((HD, E_PAD), _f32)] * 2,
    )(skw, dkw, svw, dvw, skb, dkb, svb, dvb, fu, fv, qd)


def _nodewise_body(nw, nb, agg0, agg1, feat, g2, out_o):
    agg = agg0[...] + agg1[...]
    ar = jnp.concatenate([agg] * D, axis=0)
    S = _group_sum_matrix_t(D * D, D)
    pre = jnp.dot(S, nw[...] * ar, preferred_element_type=_f32) + nb[...]
    o = jnp.maximum(pre, 0.0) + feat[...]
    mu = jnp.mean(o, axis=0, keepdims=True)
    dlt = o - mu
    var = jnp.mean(dlt * dlt, axis=0, keepdims=True)
    gb = g2[...]
    out_o[...] = dlt * lax.rsqrt(var + 1e-5) * gb[:, 0:1] + gb[:, 1:2]


def _nodewise(nw_t, nb_t, agg01_t, feat_t, ln_gamma, ln_beta):
    wspec = pl.BlockSpec((D * D, BN), lambda i: (0, i))
    vspec = pl.BlockSpec((D, BN), lambda i: (0, i))
    a0spec = pl.BlockSpec((D, BN), lambda i: (0, i))
    a1spec = pl.BlockSpec((D, BN), lambda i: (0, i + N_PAD // BN))
    gspec = pl.BlockSpec((D, 128), lambda i: (0, 0))
    g2 = jnp.zeros((D, 128), _f32)
    g2 = g2.at[:, 0].set(ln_gamma).at[:, 1].set(ln_beta)
    return pl.pallas_call(
        _nodewise_body,
        grid=((N + BN - 1) // BN,),
        in_specs=[wspec, vspec, a0spec, a1spec, vspec, gspec],
        out_specs=vspec,
        out_shape=jax.ShapeDtypeStruct((D, N), _f32),
    )(nw_t, nb_t, agg01_t, agg01_t, feat_t, g2)


def kernel(feat, edge_index, query, node_weight, node_bias,
           src_key_weight, dst_key_weight, src_key_bias, dst_key_bias,
           src_value_weight, dst_value_weight, src_value_bias, dst_value_bias,
           ln_gamma, ln_beta):
    src = jnp.pad(
        jnp.pad(edge_index[0], (0, E_PAD - E)).reshape(NW, K, CHUNK),
        ((0, 0), (0, KP - K), (0, 0)))
    dst = jnp.pad(
        jnp.pad(edge_index[1], (0, E_PAD - E), constant_values=N)
        .reshape(NW, K, CHUNK),
        ((0, 0), (0, KP - K), (0, 0)))

    fu_p, fv_p, qd_p = _sc_gather(src, dst, feat, query.reshape(N, HD))

    def wv(w):
        return w.transpose(1, 2, 3, 0).reshape(HD * D, E)

    def bv(b):
        return b.transpose(1, 2, 0).reshape(HD, E)

    key_t, val_t, exr_p, val_p = _edgewise(
        wv(src_key_weight), wv(dst_key_weight),
        wv(src_value_weight), wv(dst_value_weight),
        bv(src_key_bias), bv(dst_key_bias),
        bv(src_value_bias), bv(dst_value_bias),
        fu_p, fv_p, qd_p)

    attn_t, agg01 = _sc_softmax_agg(dst, exr_p, val_p)
    attn = attn_t[:, :E].T

    out_t = _nodewise(node_weight.transpose(1, 2, 0).reshape(D * D, N),
                      node_bias.T, agg01, feat.T, ln_gamma, ln_beta)
    return (out_t.T, key_t.T, val_t.T, attn)


# speedup vs baseline: 21.4890x; 1.0126x over previous
import functools

import jax
import jax.numpy as jnp
from jax import lax
from jax.experimental import pallas as pl
from jax.experimental.pallas import tpu as pltpu
from jax.experimental.pallas import tpu_sc as plsc

N, E, H, DH, D = 10000, 50000, 4, 4, 16
HD = H * DH

BE = 2048
BN = 2048

NC, NS = 2, 16
NW = NC * NS
CHUNK = 128
K = 13
KP = 16
BPW = K * CHUNK
SPAN2 = 2 * BPW
E_PAD = NW * BPW
N_PAD = 16384
NPS = N_PAD // NS

_mesh = plsc.VectorSubcoreMesh(core_axis_name="c", subcore_axis_name="s",
                               num_cores=NC, num_subcores=NS)
_sc_params = pltpu.CompilerParams(use_tc_tiling_on_sc=False,
                                 needs_layout_passes=False)
_f32 = jnp.float32


@functools.partial(
    pl.kernel,
    out_type=[jax.ShapeDtypeStruct((D, E_PAD), _f32)] * 3,
    mesh=_mesh,
    compiler_params=_sc_params,
    scratch_types=[
        pltpu.VMEM((KP, CHUNK), jnp.int32),
        pltpu.VMEM((KP, CHUNK), jnp.int32),
        pltpu.VMEM((BPW, D), _f32),
        pltpu.VMEM((BPW, D), _f32),
        pltpu.VMEM((BPW, D), _f32),
        pltpu.VMEM((D, BPW), _f32),
        pltpu.SemaphoreType.DMA,
        pltpu.SemaphoreType.DMA,
        pltpu.SemaphoreType.DMA,
    ],
)
def _sc_gather(src3, dst3, feat_hbm, query_hbm, fu_o, fv_o, qd_o,
               sidx, didx, fub, fvb, qdb, tb, sem0, sem1, sem2):
    wid = lax.axis_index("s") * NC + lax.axis_index("c")
    pltpu.sync_copy(src3.at[wid], sidx)
    pltpu.sync_copy(dst3.at[wid], didx)

    def _start(j):
        pltpu.async_copy(feat_hbm.at[sidx.at[j]],
                         fub.at[pl.ds(j * CHUNK, CHUNK)], sem0)
        pltpu.async_copy(feat_hbm.at[didx.at[j]],
                         fvb.at[pl.ds(j * CHUNK, CHUNK)], sem1)
        pltpu.async_copy(query_hbm.at[didx.at[j]],
                         qdb.at[pl.ds(j * CHUNK, CHUNK)], sem2)

    def _wait(j):
        pltpu.make_async_copy(feat_hbm.at[sidx.at[j]],
                              fub.at[pl.ds(j * CHUNK, CHUNK)], sem0).wait()
        pltpu.make_async_copy(feat_hbm.at[didx.at[j]],
                              fvb.at[pl.ds(j * CHUNK, CHUNK)], sem1).wait()
        pltpu.make_async_copy(query_hbm.at[didx.at[j]],
                              qdb.at[pl.ds(j * CHUNK, CHUNK)], sem2).wait()

    @pl.loop(0, K)
    def _chunks(j):
        _start(j)

        @pl.when(j > 0)
        def _():
            _wait(j - 1)

    _wait(K - 1)
    base = wid * BPW
    iot = lax.iota(jnp.int32, D)

    def _emit_t(buf, out):
        @pl.loop(0, BPW // D, unroll=2)
        def _tr(b):
            rows = b * D + iot
            for c in range(D):
                col = jnp.full((D,), c, jnp.int32)
                tb[c, pl.ds(b * D, D)] = plsc.load_gather(buf, [rows, col])
        for r in range(D):
            pltpu.async_copy(tb.at[r], out.at[r, pl.ds(base, BPW)], sem0)
        for r in range(D):
            pltpu.make_async_copy(tb.at[r], out.at[r, pl.ds(base, BPW)],
                                  sem0).wait()

    _emit_t(fub, fu_o)
    _emit_t(fvb, fv_o)
    _emit_t(qdb, qd_o)


@functools.partial(
    pl.kernel,
    out_type=[jax.ShapeDtypeStruct((H, E_PAD), _f32),
              jax.ShapeDtypeStruct((D, NC * N_PAD), _f32)],
    mesh=_mesh,
    compiler_params=_sc_params,
    scratch_types=[
        pltpu.VMEM((2, KP, CHUNK), jnp.int32),
        pltpu.VMEM((HD, BPW), _f32),
        pltpu.VMEM((BPW, HD), _f32),
        pltpu.VMEM((BPW, HD), _f32),
        pltpu.VMEM((BPW, HD), _f32),
        pltpu.VMEM((CHUNK, HD), _f32),
        pltpu.VMEM_SHARED((N_PAD, HD), _f32),
        pltpu.SemaphoreType.DMA,
        pltpu.SemaphoreType.DMA,
    ],
)
def _sc_softmax_agg(dst3, exr_hbm, val_hbm, attn_o, agg_o,
                    didx, slab, exb, vb, g0, sbt, den_sh,
                    sem_s, sem_g):
    cid = lax.axis_index("c")
    sid = lax.axis_index("s")
    base_own = sid * SPAN2 + cid * BPW
    base_c = sid * SPAN2 + (1 - cid) * BPW
    pltpu.sync_copy(dst3.at[pl.ds(2 * sid, 2)], didx)

    iot = lax.iota(jnp.int32, HD)

    def _slab_load(src, base):
        for r in range(HD):
            pltpu.async_copy(src.at[r, pl.ds(base, BPW)], slab.at[r], sem_s)
        for r in range(HD):
            pltpu.make_async_copy(src.at[r, pl.ds(base, BPW)], slab.at[r],
                                  sem_s).wait()

    def _slab_to_rows(dst):
        @pl.loop(0, BPW // HD, unroll=2)
        def _tr(b):
            for r in range(HD):
                col = jnp.full((HD,), b * HD + r, jnp.int32)
                dst[b * HD + r, :] = plsc.load_gather(slab, [iot, col])

    def _rows_to_slab(src, ncols):
        @pl.loop(0, ncols // HD, unroll=2)
        def _tr(b):
            rows = b * HD + iot
            for c in range(HD):
                col = jnp.full((HD,), c, jnp.int32)
                slab[c, pl.ds(b * HD, HD)] = plsc.load_gather(src, [rows, col])

    _slab_load(exr_hbm, base_own)
    _slab_to_rows(exb)
    _slab_load(val_hbm, base_own)
    _slab_to_rows(vb)

    @pl.loop(0, CHUNK, unroll=4)
    def _zero(i):
        sbt[i, :] = jnp.zeros((HD,), _f32)

    for t in range(NPS // CHUNK):
        pltpu.sync_copy(sbt, den_sh.at[pl.ds(sid * NPS + t * CHUNK, CHUNK)])
    plsc.subcore_barrier()

    @pl.loop(0, K)
    def _scat_own(j):
        pltpu.sync_copy(exb.at[pl.ds(j * CHUNK, CHUNK)],
                        den_sh.at[didx.at[cid, j]], add=True)

    _slab_load(exr_hbm, base_c)

    @pl.loop(0, K)
    def _scat_compl(j):
        @pl.loop(0, CHUNK // HD)
        def _tr(b):
            for r in range(HD):
                col = jnp.full((HD,), j * CHUNK + b * HD + r, jnp.int32)
                sbt[b * HD + r, :] = plsc.load_gather(slab, [iot, col])

        pltpu.sync_copy(sbt, den_sh.at[didx.at[1 - cid, j]], add=True)

    plsc.subcore_barrier()

    def _gstart(j):
        pltpu.async_copy(den_sh.at[didx.at[cid, j]],
                         g0.at[pl.ds(j * CHUNK, CHUNK)], sem_g)

    def _gwait(j):
        pltpu.make_async_copy(den_sh.at[didx.at[cid, j]],
                              g0.at[pl.ds(j * CHUNK, CHUNK)], sem_g).wait()

    @pl.loop(0, K)
    def _gath(j):
        _gstart(j)

        @pl.when(j > 0)
        def _():
            _gwait(j - 1)

    _gwait(K - 1)
    plsc.subcore_barrier()

    @pl.loop(0, CHUNK, unroll=4)
    def _zero2(i):
        sbt[i, :] = jnp.zeros((HD,), _f32)

    for t in range(NPS // CHUNK):
        pltpu.sync_copy(sbt, den_sh.at[pl.ds(sid * NPS + t * CHUNK, CHUNK)])
    plsc.subcore_barrier()

    @pl.loop(0, BPW, unroll=4)
    def _norm(i):
        rt = exb[i, :] / (g0[i, :] + 1e-16)
        vb[i, :] = vb[i, :] * rt
        exb[i, :] = rt

    @pl.loop(0, BPW // HD, unroll=2)
    def _trat(b):
        rows = b * HD + iot
        for h in range(H):
            col = jnp.full((HD,), h * DH, jnp.int32)
            slab[h, pl.ds(b * HD, HD)] = plsc.load_gather(exb, [rows, col])

    for r in range(H):
        pltpu.async_copy(slab.at[r], attn_o.at[r, pl.ds(base_own, BPW)],
                         sem_s)
    for r in range(H):
        pltpu.make_async_copy(slab.at[r], attn_o.at[r, pl.ds(base_own, BPW)],
                              sem_s).wait()

    @pl.loop(0, K)
    def _scat_msg(j):
        pltpu.sync_copy(vb.at[pl.ds(j * CHUNK, CHUNK)],
                        den_sh.at[didx.at[cid, j]], add=True)

    plsc.subcore_barrier()
    pltpu.sync_copy(den_sh.at[pl.ds(sid * NPS, NPS)], exb.at[pl.ds(0, NPS)])
    _rows_to_slab(exb, NPS)
    abase = cid * N_PAD + sid * NPS
    for r in range(D):
        pltpu.async_copy(slab.at[r, pl.ds(0, NPS)],
                         agg_o.at[r, pl.ds(abase, NPS)], sem_s)
    for r in range(D):
        pltpu.make_async_copy(slab.at[r, pl.ds(0, NPS)],
                              agg_o.at[r, pl.ds(abase, NPS)], sem_s).wait()


def _group_sum_matrix_t(rows, group):
    o = lax.broadcasted_iota(jnp.int32, (rows // group, rows), 0)
    j = lax.broadcasted_iota(jnp.int32, (rows // group, rows), 1)
    return (j // group == o).astype(_f32)


def _group_sum_repl_matrix(n, group):
    i = lax.broadcasted_iota(jnp.int32, (n, n), 0)
    j = lax.broadcasted_iota(jnp.int32, (n, n), 1)
    return (i // group == j // group).astype(_f32)


def _edgewise_body(skw, dkw, svw, dvw, skb, dkb, svb, dvb, fu, fv, qd,
                   key_o, val_o, exr_o, val2_o):
    fur = jnp.concatenate([fu[...]] * HD, axis=0)
    fvr = jnp.concatenate([fv[...]] * HD, axis=0)
    S = _group_sum_matrix_t(D * HD, D)
    kp = skw[...] * fur + dkw[...] * fvr
    key = jnp.maximum(
        jnp.dot(S, kp, preferred_element_type=_f32) + skb[...] + dkb[...],
        0.0)
    vp = svw[...] * fur + dvw[...] * fvr
    val = jnp.maximum(
        jnp.dot(S, vp, preferred_element_type=_f32) + svb[...] + dvb[...],
        0.0)
    key_o[...] = key
    val_o[...] = val
    val2_o[...] = val
    G = _group_sum_repl_matrix(HD, DH)
    lr = jnp.dot(G, key * qd[...], preferred_element_type=_f32)
    exr_o[...] = jnp.exp(lr)


def _edgewise(skw, dkw, svw, dvw, skb, dkb, svb, dvb, fu, fv, qd):
    wspec = pl.BlockSpec((D * HD, BE), lambda i: (0, i))
    vspec = pl.BlockSpec((HD, BE), lambda i: (0, i))
    grid = (E + BE - 1) // BE
    return pl.pallas_call(
        _edgewise_body,
        grid=(grid,),
        in_specs=[wspec] * 4 + [vspec] * 7,
        out_specs=[vspec] * 4,
        out_shape=[jax.ShapeDtypeStruct((HD, E), _f32)] * 2
        + [jax.ShapeDtypeStruct((HD, E_PAD), _f32)] * 2,
    )(skw, dkw, svw, dvw, skb, dkb, svb, dvb, fu, fv, qd)


def _nodewise_body(nw, nb, agg0, agg1, feat, g2, out_o):
    agg = agg0[...] + agg1[...]
    ar = jnp.concatenate([agg] * D, axis=0)
    S = _group_sum_matrix_t(D * D, D)
    pre = jnp.dot(S, nw[...] * ar, preferred_element_type=_f32) + nb[...]
    o = jnp.maximum(pre, 0.0) + feat[...]
    mu = jnp.mean(o, axis=0, keepdims=True)
    dlt = o - mu
    var = jnp.mean(dlt * dlt, axis=0, keepdims=True)
    gb = g2[...]
    out_o[...] = dlt * lax.rsqrt(var + 1e-5) * gb[:, 0:1] + gb[:, 1:2]


def _nodewise(nw_t, nb_t, agg01_t, feat_t, ln_gamma, ln_beta):
    wspec = pl.BlockSpec((D * D, BN), lambda i: (0, i))
    vspec = pl.BlockSpec((D, BN), lambda i: (0, i))
    a0spec = pl.BlockSpec((D, BN), lambda i: (0, i))
    a1spec = pl.BlockSpec((D, BN), lambda i: (0, i + N_PAD // BN))
    gspec = pl.BlockSpec((D, 128), lambda i: (0, 0))
    g2 = jnp.zeros((D, 128), _f32)
    g2 = g2.at[:, 0].set(ln_gamma).at[:, 1].set(ln_beta)
    return pl.pallas_call(
        _nodewise_body,
        grid=((N + BN - 1) // BN,),
        in_specs=[wspec, vspec, a0spec, a1spec, vspec, gspec],
        out_specs=vspec,
        out_shape=jax.ShapeDtypeStruct((D, N), _f32),
    )(nw_t, nb_t, agg01_t, agg01_t, feat_t, g2)


def kernel(feat, edge_index, query, node_weight, node_bias,
           src_key_weight, dst_key_weight, src_key_bias, dst_key_bias,
           src_value_weight, dst_value_weight, src_value_bias, dst_value_bias,
           ln_gamma, ln_beta):
    src = jnp.pad(
        jnp.pad(edge_index[0], (0, E_PAD - E)).reshape(NW, K, CHUNK),
        ((0, 0), (0, KP - K), (0, 0)))
    dst = jnp.pad(
        jnp.pad(edge_index[1], (0, E_PAD - E), constant_values=N)
        .reshape(NW, K, CHUNK),
        ((0, 0), (0, KP - K), (0, 0)))

    fu_p, fv_p, qd_p = _sc_gather(src, dst, feat, query.reshape(N, HD))

    def wv(w):
        return w.transpose(1, 2, 3, 0).reshape(HD * D, E)

    def bv(b):
        return b.transpose(1, 2, 0).reshape(HD, E)

    key_t, val_t, exr_p, val_p = _edgewise(
        wv(src_key_weight), wv(dst_key_weight),
        wv(src_value_weight), wv(dst_value_weight),
        bv(src_key_bias), bv(dst_key_bias),
        bv(src_value_bias), bv(dst_value_bias),
        fu_p, fv_p, qd_p)

    attn_t, agg01 = _sc_softmax_agg(dst, exr_p, val_p)
    attn = attn_t[:, :E].T

    out_t = _nodewise(node_weight.transpose(1, 2, 0).reshape(D * D, N),
                      node_bias.T, agg01, feat.T, ln_gamma, ln_beta)
    return (out_t.T, key_t.T, val_t.T, attn)
